# baseline (speedup 1.0000x reference)
import functools

import jax
import jax.numpy as jnp
from jax import lax
from jax.experimental import pallas as pl
from jax.experimental.pallas import tpu as pltpu

DIM = 96
HEADS = 4
H = W = 384
L = H * W
HW4 = L // HEADS
CPH = DIM // HEADS


_HB = 8
_NHB = H // _HB


def _qkv_body(xa_ref, xb_ref, w_ref, dw_ref, out_ref):
    x16 = jnp.concatenate([xa_ref[...], xb_ref[...]], axis=1)
    xh = x16[:, 0:_HB + 2, :]
    mm = jnp.dot(w_ref[...], xh.reshape(DIM, -1),
                 preferred_element_type=jnp.float32,
                 precision=lax.Precision.HIGHEST)
    qh = mm.reshape(DIM, _HB + 2, W + 2)
    acc = jnp.zeros((DIM, _HB, W), dtype=jnp.float32)
    for di in range(3):
        for dj in range(3):
            tap = dw_ref[:, 3 * di + dj][:, None, None]
            acc = acc + qh[:, di:di + _HB, dj:dj + W] * tap
    out_ref[...] = acc


def _dw_body(xa_ref, xb_ref, dw_ref, out_ref):
    x16 = jnp.concatenate([xa_ref[...], xb_ref[...]], axis=1)
    qh = x16[:, 0:_HB + 2, :].astype(jnp.bfloat16).astype(jnp.float32)
    dw = dw_ref[...].astype(jnp.bfloat16).astype(jnp.float32)
    acc = jnp.zeros((DIM, _HB, W), dtype=jnp.float32)
    for di in range(3):
        for dj in range(3):
            tap = dw[:, 3 * di + dj][:, None, None]
            acc = acc + qh[:, di:di + _HB, dj:dj + W] * tap
    out_ref[...] = acc


def _dw_conv(q_pad, dw2d):
    return pl.pallas_call(
        _dw_body,
        grid=(5, _NHB),
        in_specs=[
            pl.BlockSpec((DIM, _HB, W + 2), lambda cb, hb: (cb, hb, 0)),
            pl.BlockSpec((DIM, _HB, W + 2), lambda cb, hb: (cb, hb + 1, 0)),
            pl.BlockSpec((DIM, 9), lambda cb, hb: (cb, 0)),
        ],
        out_specs=pl.BlockSpec((DIM, _HB, W), lambda cb, hb: (cb, hb, 0)),
        out_shape=jax.ShapeDtypeStruct((5 * DIM, H, W), jnp.float32),
    )(q_pad, q_pad, dw2d)


def _qkv_conv(x_pad, w2d, dw2d):
    grid = (5, _NHB)
    return pl.pallas_call(
        _qkv_body,
        grid=grid,
        in_specs=[
            pl.BlockSpec((DIM, _HB, W + 2), lambda cb, hb: (0, hb, 0)),
            pl.BlockSpec((DIM, _HB, W + 2), lambda cb, hb: (0, hb + 1, 0)),
            pl.BlockSpec((DIM, DIM), lambda cb, hb: (cb, 0)),
            pl.BlockSpec((DIM, 9), lambda cb, hb: (cb, 0)),
        ],
        out_specs=pl.BlockSpec((DIM, _HB, W), lambda cb, hb: (cb, hb, 0)),
        out_shape=jax.ShapeDtypeStruct((5 * DIM, H, W), jnp.float32),
    )(x_pad, x_pad, w2d, dw2d)


_PC = 4096
_NPC = HW4 // _PC


def _gram_body(qk_ref, g_ref):
    @pl.when(pl.program_id(1) == 0)
    def _():
        g_ref[...] = jnp.zeros_like(g_ref)
    qk = qk_ref[0]
    g_ref[0] += jnp.dot(qk, qk.T, preferred_element_type=jnp.float32)


def _gram(qk):
    return pl.pallas_call(
        _gram_body,
        grid=(HEADS, _NPC),
        in_specs=[pl.BlockSpec((1, 2 * DIM, _PC), lambda h, p: (h, 0, p))],
        out_specs=pl.BlockSpec((1, 2 * DIM, 2 * DIM), lambda h, p: (h, 0, 0)),
        out_shape=jax.ShapeDtypeStruct((HEADS, 2 * DIM, 2 * DIM), jnp.float32),
    )(qk)


def _attn_body(g_ref, t_ref, a_ref):
    g = g_ref[0]
    n = 2 * DIM
    eye = (lax.broadcasted_iota(jnp.int32, (n, n), 0)
           == lax.broadcasted_iota(jnp.int32, (n, n), 1)).astype(jnp.float32)
    diag = jnp.sum(g * eye, axis=1)
    inv = 1.0 / jnp.maximum(jnp.sqrt(diag), 1e-12)
    sim = g[:DIM, DIM:] * inv[:DIM, None] * inv[None, DIM:]
    t = t_ref[0][0:1, 0:1]
    e = jnp.exp(sim * t)
    a_ref[0] = e / (jnp.sum(e, axis=1, keepdims=True) + 1.0)


def _attn_softmax(g, temp_b):
    return pl.pallas_call(
        _attn_body,
        grid=(HEADS,),
        in_specs=[
            pl.BlockSpec((1, 2 * DIM, 2 * DIM), lambda h: (h, 0, 0)),
            pl.BlockSpec((1, 8, 128), lambda h: (h, 0, 0)),
        ],
        out_specs=pl.BlockSpec((1, DIM, DIM), lambda h: (h, 0, 0)),
        out_shape=jax.ShapeDtypeStruct((HEADS, DIM, DIM), jnp.float32),
    )(g, temp_b)


def _mix_body(a_ref, v_ref, o_ref):
    o_ref[0] = jnp.dot(a_ref[0], v_ref[0], preferred_element_type=jnp.float32)


def _mix(attn, v):
    return pl.pallas_call(
        _mix_body,
        grid=(HEADS, _NPC),
        in_specs=[
            pl.BlockSpec((1, DIM, DIM), lambda h, p: (h, 0, 0)),
            pl.BlockSpec((1, DIM, _PC), lambda h, p: (h, 0, p)),
        ],
        out_specs=pl.BlockSpec((1, DIM, _PC), lambda h, p: (h, 0, p)),
        out_shape=jax.ShapeDtypeStruct((HEADS, DIM, HW4), jnp.float32),
    )(attn, v)


_LC = 8192
_NLC = L // _LC


def _proj_body(w_ref, x_ref, o_ref):
    o_ref[...] = jnp.dot(w_ref[...], x_ref[...],
                         preferred_element_type=jnp.float32)


def _proj(w2d, x2d):
    o, i = w2d.shape
    return pl.pallas_call(
        _proj_body,
        grid=(_NLC,),
        in_specs=[
            pl.BlockSpec((o, i), lambda j: (0, 0)),
            pl.BlockSpec((i, _LC), lambda j: (0, j)),
        ],
        out_specs=pl.BlockSpec((o, _LC), lambda j: (0, j)),
        out_shape=jax.ShapeDtypeStruct((o, L), jnp.float32),
    )(w2d, x2d)


def _scatter_axis(idx, vals, axis):
    grids = list(jnp.indices(idx.shape))
    grids[axis] = idx
    return jnp.zeros_like(vals).at[tuple(grids)].set(vals)


def _fold_box(t):
    return t.reshape(HEADS, CPH, HEADS, HW4).reshape(HEADS, DIM, HW4)


def _unfold_box(t):
    return t.reshape(HEADS, CPH, HEADS, HW4).reshape(DIM, L)


def _perm_nb():
    return (jnp.arange(HW4, dtype=jnp.int32)[None, :] * HEADS
            + jnp.arange(HEADS, dtype=jnp.int32)[:, None]).reshape(L)


def _iperm_nb():
    return (jnp.arange(HEADS, dtype=jnp.int32)[None, :] * HW4
            + jnp.arange(HW4, dtype=jnp.int32)[:, None]).reshape(L)


def kernel(x, w_qkv, w_dw, w_out, temperature):
    xs = x[0]
    half = DIM // 2

    xh = xs[:half]
    idx_h = jnp.argsort(xh, axis=-2)
    x_sort = jnp.take_along_axis(xh, idx_h, axis=-2)
    idx_w = jnp.argsort(x_sort, axis=-1)
    x_sort = jnp.take_along_axis(x_sort, idx_w, axis=-1)
    xs = xs.at[:half].set(x_sort)

    _c = _proj(w_qkv[:, :, 0, 0], xs.reshape(DIM, L)).reshape(5 * DIM, H, W)
    qkv = jax.lax.conv_general_dilated(
        _c[None], w_dw, window_strides=(1, 1), padding='SAME',
        feature_group_count=5 * DIM,
        dimension_numbers=('NCHW', 'OIHW', 'NCHW'))[0]
    q1, k1, q2, k2, v = jnp.split(qkv.reshape(5, DIM, L), 5, axis=0)
    q1, k1, q2, k2, v = q1[0], k1[0], q2[0], k2[0], v[0]

    idx = jnp.argsort(v, axis=-1)
    vs = jnp.take_along_axis(v, idx, axis=-1)
    idx2 = jnp.take(idx, _perm_nb(), axis=1)
    g = lambda t: jnp.take_along_axis(t, idx, axis=-1)
    g2 = lambda t: jnp.take_along_axis(t, idx2, axis=-1)
    q1s, k1s = g(q1), g(k1)
    q2s_nb, k2s_nb, vs_nb = g2(q2), g2(k2), g2(v)

    temp_b = jnp.broadcast_to(temperature.reshape(HEADS, 1, 1), (HEADS, 8, 128))

    qk1 = jnp.concatenate([_fold_box(q1s), _fold_box(k1s)], axis=1)
    attn1 = _attn_softmax(_gram(qk1), temp_b)
    out1 = _mix(attn1, _fold_box(vs))

    qk2 = jnp.concatenate([_fold_box(q2s_nb), _fold_box(k2s_nb)], axis=1)
    attn2 = _attn_softmax(_gram(qk2), temp_b)
    out2 = _mix(attn2, _fold_box(vs_nb))

    out2n = jnp.take(_unfold_box(out2), _iperm_nb(), axis=1)
    prod = _unfold_box(out1) * out2n
    res = _scatter_axis(idx, prod, 1)

    out = _proj(w_out[:, :, 0, 0], res)
    out = out.reshape(DIM, H, W)

    orp = out[:half]
    orp = _scatter_axis(idx_w, orp, 2)
    orp = _scatter_axis(idx_h, orp, 1)
    out = out.at[:half].set(orp)
    return out[None]


# speedup vs baseline: 1.0434x; 1.0434x over previous
import functools

import jax
import jax.numpy as jnp
from jax import lax
from jax.experimental import pallas as pl
from jax.experimental.pallas import tpu as pltpu
from jax.experimental.pallas import tpu_sc as plsc

DIM = 96
HEADS = 4
H = W = 384
L = H * W
HW4 = L // HEADS
CPH = DIM // HEADS


_HB = 8
_NHB = H // _HB


def _qkv_body(xa_ref, xb_ref, w_ref, dw_ref, out_ref):
    x16 = jnp.concatenate([xa_ref[...], xb_ref[...]], axis=1)
    xh = x16[:, 0:_HB + 2, :]
    mm = jnp.dot(w_ref[...], xh.reshape(DIM, -1),
                 preferred_element_type=jnp.float32,
                 precision=lax.Precision.HIGHEST)
    qh = mm.reshape(DIM, _HB + 2, W + 2)
    acc = jnp.zeros((DIM, _HB, W), dtype=jnp.float32)
    for di in range(3):
        for dj in range(3):
            tap = dw_ref[:, 3 * di + dj][:, None, None]
            acc = acc + qh[:, di:di + _HB, dj:dj + W] * tap
    out_ref[...] = acc


def _dw_body(xa_ref, xb_ref, dw_ref, out_ref):
    x16 = jnp.concatenate([xa_ref[...], xb_ref[...]], axis=1)
    qh = x16[:, 0:_HB + 2, :].astype(jnp.bfloat16).astype(jnp.float32)
    dw = dw_ref[...].astype(jnp.bfloat16).astype(jnp.float32)
    acc = jnp.zeros((DIM, _HB, W), dtype=jnp.float32)
    for di in range(3):
        for dj in range(3):
            tap = dw[:, 3 * di + dj][:, None, None]
            acc = acc + qh[:, di:di + _HB, dj:dj + W] * tap
    out_ref[...] = acc


def _dw_conv(q_pad, dw2d):
    return pl.pallas_call(
        _dw_body,
        grid=(5, _NHB),
        in_specs=[
            pl.BlockSpec((DIM, _HB, W + 2), lambda cb, hb: (cb, hb, 0)),
            pl.BlockSpec((DIM, _HB, W + 2), lambda cb, hb: (cb, hb + 1, 0)),
            pl.BlockSpec((DIM, 9), lambda cb, hb: (cb, 0)),
        ],
        out_specs=pl.BlockSpec((DIM, _HB, W), lambda cb, hb: (cb, hb, 0)),
        out_shape=jax.ShapeDtypeStruct((5 * DIM, H, W), jnp.float32),
    )(q_pad, q_pad, dw2d)


def _qkv_conv(x_pad, w2d, dw2d):
    grid = (5, _NHB)
    return pl.pallas_call(
        _qkv_body,
        grid=grid,
        in_specs=[
            pl.BlockSpec((DIM, _HB, W + 2), lambda cb, hb: (0, hb, 0)),
            pl.BlockSpec((DIM, _HB, W + 2), lambda cb, hb: (0, hb + 1, 0)),
            pl.BlockSpec((DIM, DIM), lambda cb, hb: (cb, 0)),
            pl.BlockSpec((DIM, 9), lambda cb, hb: (cb, 0)),
        ],
        out_specs=pl.BlockSpec((DIM, _HB, W), lambda cb, hb: (cb, hb, 0)),
        out_shape=jax.ShapeDtypeStruct((5 * DIM, H, W), jnp.float32),
    )(x_pad, x_pad, w2d, dw2d)


_PC = 4096
_NPC = HW4 // _PC


def _gram_body(qk_ref, g_ref):
    @pl.when(pl.program_id(1) == 0)
    def _():
        g_ref[...] = jnp.zeros_like(g_ref)
    qk = qk_ref[0]
    g_ref[0] += jnp.dot(qk, qk.T, preferred_element_type=jnp.float32)


def _gram(qk):
    return pl.pallas_call(
        _gram_body,
        grid=(HEADS, _NPC),
        in_specs=[pl.BlockSpec((1, 2 * DIM, _PC), lambda h, p: (h, 0, p))],
        out_specs=pl.BlockSpec((1, 2 * DIM, 2 * DIM), lambda h, p: (h, 0, 0)),
        out_shape=jax.ShapeDtypeStruct((HEADS, 2 * DIM, 2 * DIM), jnp.float32),
    )(qk)


def _attn_body(g_ref, t_ref, a_ref):
    g = g_ref[0]
    n = 2 * DIM
    eye = (lax.broadcasted_iota(jnp.int32, (n, n), 0)
           == lax.broadcasted_iota(jnp.int32, (n, n), 1)).astype(jnp.float32)
    diag = jnp.sum(g * eye, axis=1)
    inv = 1.0 / jnp.maximum(jnp.sqrt(diag), 1e-12)
    sim = g[:DIM, DIM:] * inv[:DIM, None] * inv[None, DIM:]
    t = t_ref[0][0:1, 0:1]
    e = jnp.exp(sim * t)
    a_ref[0] = e / (jnp.sum(e, axis=1, keepdims=True) + 1.0)


def _attn_softmax(g, temp_b):
    return pl.pallas_call(
        _attn_body,
        grid=(HEADS,),
        in_specs=[
            pl.BlockSpec((1, 2 * DIM, 2 * DIM), lambda h: (h, 0, 0)),
            pl.BlockSpec((1, 8, 128), lambda h: (h, 0, 0)),
        ],
        out_specs=pl.BlockSpec((1, DIM, DIM), lambda h: (h, 0, 0)),
        out_shape=jax.ShapeDtypeStruct((HEADS, DIM, DIM), jnp.float32),
    )(g, temp_b)


def _mix_body(a_ref, v_ref, o_ref):
    o_ref[0] = jnp.dot(a_ref[0], v_ref[0], preferred_element_type=jnp.float32)


def _mix(attn, v):
    return pl.pallas_call(
        _mix_body,
        grid=(HEADS, _NPC),
        in_specs=[
            pl.BlockSpec((1, DIM, DIM), lambda h, p: (h, 0, 0)),
            pl.BlockSpec((1, DIM, _PC), lambda h, p: (h, 0, p)),
        ],
        out_specs=pl.BlockSpec((1, DIM, _PC), lambda h, p: (h, 0, p)),
        out_shape=jax.ShapeDtypeStruct((HEADS, DIM, HW4), jnp.float32),
    )(attn, v)


_LC = 8192
_NLC = L // _LC


def _proj_body(w_ref, x_ref, o_ref):
    o_ref[...] = jnp.dot(w_ref[...], x_ref[...],
                         preferred_element_type=jnp.float32)


def _proj(w2d, x2d):
    o, i = w2d.shape
    return pl.pallas_call(
        _proj_body,
        grid=(_NLC,),
        in_specs=[
            pl.BlockSpec((o, i), lambda j: (0, 0)),
            pl.BlockSpec((i, _LC), lambda j: (0, j)),
        ],
        out_specs=pl.BlockSpec((o, _LC), lambda j: (0, j)),
        out_shape=jax.ShapeDtypeStruct((o, L), jnp.float32),
    )(w2d, x2d)


_NROW = DIM
_NSC = 2
_NTILE = 16
_CHUNK = L // _NTILE
_SUB = _CHUNK // 16
_RADIX = 256


def _sc_sort_body(v_hbm, vs_hbm, idx_hbm,
                  fbuf, keybuf, ibuf, posbuf, h2, run2, b2, hgall, histb,
                  baseb, ak, ai, bk, bi, hg, sem1, sem2):
    core = lax.axis_index("c")
    t = lax.axis_index("s")
    lanes = lax.iota(jnp.int32, 16)
    gidx0 = lanes * _SUB
    cbase = t * _CHUNK
    minint = jnp.int32(-2147483648)

    def zero_counts(ref):
        def zb(j, _):
            ref[pl.ds(j * 16, 16)] = jnp.zeros((16,), jnp.int32)
            return 0
        lax.fori_loop(0, _RADIX, zb, 0)

    def sweep_hist(sh):
        zero_counts(h2)

        def body(j, _):
            kv = plsc.load_gather(keybuf, [gidx0 + j])
            d = lax.shift_right_logical(kv, sh) & 255
            cidx = d * 16 + lanes
            h = plsc.load_gather(h2, [cidx])
            plsc.store_scatter(h2, [cidx], h + 1)
            return 0
        lax.fori_loop(0, _SUB, body, 0)
        def tot(g, _):
            dd = lanes + jnp.full((16,), g * 16, jnp.int32)
            def lsum(l, acc):
                return acc + plsc.load_gather(h2, [dd * 16 + l])
            acc = lax.fori_loop(0, 16, lsum, jnp.zeros((16,), jnp.int32))
            histb[pl.ds(g * 16, 16)] = acc
            return 0
        lax.fori_loop(0, _RADIX // 16, tot, 0)
        def expref(d, _):
            row = h2[pl.ds(d * 16, 16)]
            h2[pl.ds(d * 16, 16)] = plsc.cumsum(row) - row
            return 0
        lax.fori_loop(0, _RADIX, expref, 0)

    def combine():
        pltpu.sync_copy(hg, hgall)
        def dig(g, carry):
            acc = jnp.zeros((16,), jnp.int32)
            mine = jnp.zeros((16,), jnp.int32)
            def tt_body(tt, c):
                acc, mine = c
                row = hgall[tt, pl.ds(g * 16, 16)]
                mine = jnp.where(jnp.full((16,), tt, jnp.int32)
                                 < jnp.full((16,), t, jnp.int32),
                                 mine + row, mine)
                return (acc + row, mine)
            acc, mine = lax.fori_loop(0, _NTILE, tt_body, (acc, mine))
            ex = plsc.cumsum(acc) - acc + jnp.full((16,), carry, jnp.int32)
            baseb[pl.ds(g * 16, 16)] = ex + mine
            return carry + jnp.sum(acc, axis=0)
        lax.fori_loop(0, _RADIX // 16, dig, jnp.int32(0))
        def bd(g, _):
            dd = lanes + jnp.full((16,), g * 16, jnp.int32)
            bv = baseb[pl.ds(g * 16, 16)]
            def lb(l, _):
                cidx = dd * 16 + l
                plsc.store_scatter(b2, [cidx],
                                   plsc.load_gather(h2, [cidx]) + bv)
                return 0
            lax.fori_loop(0, 16, lb, 0)
            return 0
        lax.fori_loop(0, _RADIX // 16, bd, 0)

    def sweep_rank(sh):
        zero_counts(run2)

        def body(j, _):
            kv = plsc.load_gather(keybuf, [gidx0 + j])
            d = lax.shift_right_logical(kv, sh) & 255
            cidx = d * 16 + lanes
            c = plsc.load_gather(run2, [cidx])
            b = plsc.load_gather(b2, [cidx])
            plsc.store_scatter(run2, [cidx], c + 1)
            plsc.store_scatter(posbuf, [gidx0 + j], b + c)
            return 0
        lax.fori_loop(0, _SUB, body, 0)

    def one_pass(sh, src_k, src_i, dst_k, dst_i, first, row):
        if first:
            pltpu.sync_copy(v_hbm.at[row, pl.ds(cbase, _CHUNK)], fbuf)
            def keyb(j, _):
                x = fbuf[pl.ds(j * 16, 16)]
                k = lax.bitcast_convert_type(x, jnp.int32)
                m = lax.shift_right_arithmetic(k, 31)
                keybuf[pl.ds(j * 16, 16)] = k ^ (m | minint)
                ibuf[pl.ds(j * 16, 16)] = lanes + jnp.full(
                    (16,), cbase + j * 16, jnp.int32)
                return 0
            lax.fori_loop(0, _SUB, keyb, 0)
        else:
            pltpu.sync_copy(src_k.at[pl.ds(cbase, _CHUNK)], keybuf)
            pltpu.sync_copy(src_i.at[pl.ds(cbase, _CHUNK)], ibuf)
        sweep_hist(sh)
        pltpu.sync_copy(histb, hg.at[t])
        plsc.subcore_barrier()
        combine()
        sweep_rank(sh)
        d1 = pltpu.async_copy(keybuf, dst_k.at[posbuf], sem1)
        d2 = pltpu.async_copy(ibuf, dst_i.at[posbuf], sem2)
        d1.wait()
        d2.wait()
        plsc.subcore_barrier()

    def do_row(r, _):
        row = core * (_NROW // _NSC) + r
        one_pass(jnp.int32(0), None, None, bk, bi, True, row)
        one_pass(jnp.int32(8), bk, bi, ak, ai, False, row)
        one_pass(jnp.int32(16), ak, ai, bk, bi, False, row)
        one_pass(jnp.int32(24), bk, bi, ak, ai, False, row)
        pltpu.sync_copy(ak.at[pl.ds(cbase, _CHUNK)], keybuf)
        pltpu.sync_copy(ai.at[pl.ds(cbase, _CHUNK)], ibuf)
        def unk(j, _):
            kv = keybuf[pl.ds(j * 16, 16)]
            m = lax.shift_right_arithmetic(kv, 31)
            orig = kv ^ jnp.where(m != 0, minint, jnp.int32(-1))
            fbuf[pl.ds(j * 16, 16)] = lax.bitcast_convert_type(orig, jnp.float32)
            return 0
        lax.fori_loop(0, _SUB, unk, 0)
        pltpu.sync_copy(fbuf, vs_hbm.at[row, pl.ds(cbase, _CHUNK)])
        pltpu.sync_copy(ibuf, idx_hbm.at[row, pl.ds(cbase, _CHUNK)])
        plsc.subcore_barrier()
        return 0

    lax.fori_loop(0, _NROW // _NSC, do_row, 0)


def _sc_argsort(v):
    mesh = plsc.VectorSubcoreMesh(core_axis_name="c", subcore_axis_name="s")
    f = pl.kernel(
        _sc_sort_body,
        out_type=(jax.ShapeDtypeStruct((_NROW, L), jnp.float32),
                  jax.ShapeDtypeStruct((_NROW, L), jnp.int32)),
        mesh=mesh,
        compiler_params=pltpu.CompilerParams(needs_layout_passes=False),
        scratch_types=[
            pltpu.VMEM((_CHUNK,), jnp.float32),
            pltpu.VMEM((_CHUNK,), jnp.int32),
            pltpu.VMEM((_CHUNK,), jnp.int32),
            pltpu.VMEM((_CHUNK,), jnp.int32),
            pltpu.VMEM((_RADIX * 16,), jnp.int32),
            pltpu.VMEM((_RADIX * 16,), jnp.int32),
            pltpu.VMEM((_RADIX * 16,), jnp.int32),
            pltpu.VMEM((_NTILE, _RADIX), jnp.int32),
            pltpu.VMEM((_RADIX,), jnp.int32),
            pltpu.VMEM((_RADIX,), jnp.int32),
            pltpu.VMEM_SHARED((L,), jnp.int32),
            pltpu.VMEM_SHARED((L,), jnp.int32),
            pltpu.VMEM_SHARED((L,), jnp.int32),
            pltpu.VMEM_SHARED((L,), jnp.int32),
            pltpu.VMEM_SHARED((_NTILE, _RADIX), jnp.int32),
            pltpu.SemaphoreType.DMA,
            pltpu.SemaphoreType.DMA,
        ])
    return f(v)


def _scatter_axis(idx, vals, axis):
    grids = list(jnp.indices(idx.shape))
    grids[axis] = idx
    return jnp.zeros_like(vals).at[tuple(grids)].set(vals)


def _fold_box(t):
    return t.reshape(HEADS, CPH, HEADS, HW4).reshape(HEADS, DIM, HW4)


def _unfold_box(t):
    return t.reshape(HEADS, CPH, HEADS, HW4).reshape(DIM, L)


def _perm_nb():
    return (jnp.arange(HW4, dtype=jnp.int32)[None, :] * HEADS
            + jnp.arange(HEADS, dtype=jnp.int32)[:, None]).reshape(L)


def _iperm_nb():
    return (jnp.arange(HEADS, dtype=jnp.int32)[None, :] * HW4
            + jnp.arange(HW4, dtype=jnp.int32)[:, None]).reshape(L)


def kernel(x, w_qkv, w_dw, w_out, temperature):
    xs = x[0]
    half = DIM // 2

    xh = xs[:half]
    idx_h = jnp.argsort(xh, axis=-2)
    x_sort = jnp.take_along_axis(xh, idx_h, axis=-2)
    idx_w = jnp.argsort(x_sort, axis=-1)
    x_sort = jnp.take_along_axis(x_sort, idx_w, axis=-1)
    xs = xs.at[:half].set(x_sort)

    _c = _proj(w_qkv[:, :, 0, 0], xs.reshape(DIM, L)).reshape(5 * DIM, H, W)
    qkv = jax.lax.conv_general_dilated(
        _c[None], w_dw, window_strides=(1, 1), padding='SAME',
        feature_group_count=5 * DIM,
        dimension_numbers=('NCHW', 'OIHW', 'NCHW'))[0]
    q1, k1, q2, k2, v = jnp.split(qkv.reshape(5, DIM, L), 5, axis=0)
    q1, k1, q2, k2, v = q1[0], k1[0], q2[0], k2[0], v[0]

    vs, idx = _sc_argsort(v)
    idx2 = jnp.take(idx, _perm_nb(), axis=1)
    g = lambda t: jnp.take_along_axis(t, idx, axis=-1)
    g2 = lambda t: jnp.take_along_axis(t, idx2, axis=-1)
    q1s, k1s = g(q1), g(k1)
    q2s_nb, k2s_nb, vs_nb = g2(q2), g2(k2), g2(v)

    temp_b = jnp.broadcast_to(temperature.reshape(HEADS, 1, 1), (HEADS, 8, 128))

    qk1 = jnp.concatenate([_fold_box(q1s), _fold_box(k1s)], axis=1)
    attn1 = _attn_softmax(_gram(qk1), temp_b)
    out1 = _mix(attn1, _fold_box(vs))

    qk2 = jnp.concatenate([_fold_box(q2s_nb), _fold_box(k2s_nb)], axis=1)
    attn2 = _attn_softmax(_gram(qk2), temp_b)
    out2 = _mix(attn2, _fold_box(vs_nb))

    out2n = jnp.take(_unfold_box(out2), _iperm_nb(), axis=1)
    prod = _unfold_box(out1) * out2n
    res = _scatter_axis(idx, prod, 1)

    out = _proj(w_out[:, :, 0, 0], res)
    out = out.reshape(DIM, H, W)

    orp = out[:half]
    orp = _scatter_axis(idx_w, orp, 2)
    orp = _scatter_axis(idx_h, orp, 1)
    out = out.at[:half].set(orp)
    return out[None]


# speedup vs baseline: 1.6469x; 1.5784x over previous
import functools

import jax
import jax.numpy as jnp
from jax import lax
from jax.experimental import pallas as pl
from jax.experimental.pallas import tpu as pltpu
from jax.experimental.pallas import tpu_sc as plsc

DIM = 96
HEADS = 4
H = W = 384
L = H * W
HW4 = L // HEADS
CPH = DIM // HEADS


_HB = 8
_NHB = H // _HB


def _qkv_body(xa_ref, xb_ref, w_ref, dw_ref, out_ref):
    x16 = jnp.concatenate([xa_ref[...], xb_ref[...]], axis=1)
    xh = x16[:, 0:_HB + 2, :]
    mm = jnp.dot(w_ref[...], xh.reshape(DIM, -1),
                 preferred_element_type=jnp.float32,
                 precision=lax.Precision.HIGHEST)
    qh = mm.reshape(DIM, _HB + 2, W + 2)
    acc = jnp.zeros((DIM, _HB, W), dtype=jnp.float32)
    for di in range(3):
        for dj in range(3):
            tap = dw_ref[:, 3 * di + dj][:, None, None]
            acc = acc + qh[:, di:di + _HB, dj:dj + W] * tap
    out_ref[...] = acc


def _dw_body(xa_ref, xb_ref, dw_ref, out_ref):
    x16 = jnp.concatenate([xa_ref[...], xb_ref[...]], axis=1)
    qh = x16[:, 0:_HB + 2, :].astype(jnp.bfloat16).astype(jnp.float32)
    dw = dw_ref[...]
    acc = jnp.zeros((DIM, _HB, W), dtype=jnp.float32)
    for di in range(3):
        for dj in range(3):
            tap = dw[:, 3 * di + dj][:, None, None]
            acc = acc + qh[:, di:di + _HB, dj:dj + W] * tap
    out_ref[...] = acc


def _dw_conv(q_pad, dw2d):
    return pl.pallas_call(
        _dw_body,
        grid=(5, _NHB),
        in_specs=[
            pl.BlockSpec((DIM, _HB, W + 2), lambda cb, hb: (cb, hb, 0)),
            pl.BlockSpec((DIM, _HB, W + 2), lambda cb, hb: (cb, hb + 1, 0)),
            pl.BlockSpec((DIM, 9), lambda cb, hb: (cb, 0)),
        ],
        out_specs=pl.BlockSpec((DIM, _HB, W), lambda cb, hb: (cb, hb, 0)),
        out_shape=jax.ShapeDtypeStruct((5 * DIM, H, W), jnp.float32),
    )(q_pad, q_pad, dw2d)


def _qkv_conv(x_pad, w2d, dw2d):
    grid = (5, _NHB)
    return pl.pallas_call(
        _qkv_body,
        grid=grid,
        in_specs=[
            pl.BlockSpec((DIM, _HB, W + 2), lambda cb, hb: (0, hb, 0)),
            pl.BlockSpec((DIM, _HB, W + 2), lambda cb, hb: (0, hb + 1, 0)),
            pl.BlockSpec((DIM, DIM), lambda cb, hb: (cb, 0)),
            pl.BlockSpec((DIM, 9), lambda cb, hb: (cb, 0)),
        ],
        out_specs=pl.BlockSpec((DIM, _HB, W), lambda cb, hb: (cb, hb, 0)),
        out_shape=jax.ShapeDtypeStruct((5 * DIM, H, W), jnp.float32),
    )(x_pad, x_pad, w2d, dw2d)


_PC = 4096
_NPC = HW4 // _PC


def _gram_body(qk_ref, g_ref):
    @pl.when(pl.program_id(1) == 0)
    def _():
        g_ref[...] = jnp.zeros_like(g_ref)
    qk = qk_ref[0]
    g_ref[0] += jnp.dot(qk, qk.T, preferred_element_type=jnp.float32)


def _gram(qk):
    return pl.pallas_call(
        _gram_body,
        grid=(HEADS, _NPC),
        in_specs=[pl.BlockSpec((1, 2 * DIM, _PC), lambda h, p: (h, 0, p))],
        out_specs=pl.BlockSpec((1, 2 * DIM, 2 * DIM), lambda h, p: (h, 0, 0)),
        out_shape=jax.ShapeDtypeStruct((HEADS, 2 * DIM, 2 * DIM), jnp.float32),
    )(qk)


def _attn_body(g_ref, t_ref, a_ref):
    g = g_ref[0]
    n = 2 * DIM
    eye = (lax.broadcasted_iota(jnp.int32, (n, n), 0)
           == lax.broadcasted_iota(jnp.int32, (n, n), 1)).astype(jnp.float32)
    diag = jnp.sum(g * eye, axis=1)
    inv = 1.0 / jnp.maximum(jnp.sqrt(diag), 1e-12)
    sim = g[:DIM, DIM:] * inv[:DIM, None] * inv[None, DIM:]
    t = t_ref[0][0:1, 0:1]
    e = jnp.exp(sim * t)
    a_ref[0] = e / (jnp.sum(e, axis=1, keepdims=True) + 1.0)


def _attn_softmax(g, temp_b):
    return pl.pallas_call(
        _attn_body,
        grid=(HEADS,),
        in_specs=[
            pl.BlockSpec((1, 2 * DIM, 2 * DIM), lambda h: (h, 0, 0)),
            pl.BlockSpec((1, 8, 128), lambda h: (h, 0, 0)),
        ],
        out_specs=pl.BlockSpec((1, DIM, DIM), lambda h: (h, 0, 0)),
        out_shape=jax.ShapeDtypeStruct((HEADS, DIM, DIM), jnp.float32),
    )(g, temp_b)


def _mix_body(a_ref, v_ref, o_ref):
    o_ref[0] = jnp.dot(a_ref[0], v_ref[0], preferred_element_type=jnp.float32)


def _mix(attn, v):
    return pl.pallas_call(
        _mix_body,
        grid=(HEADS, _NPC),
        in_specs=[
            pl.BlockSpec((1, DIM, DIM), lambda h, p: (h, 0, 0)),
            pl.BlockSpec((1, DIM, _PC), lambda h, p: (h, 0, p)),
        ],
        out_specs=pl.BlockSpec((1, DIM, _PC), lambda h, p: (h, 0, p)),
        out_shape=jax.ShapeDtypeStruct((HEADS, DIM, HW4), jnp.float32),
    )(attn, v)


_LC = 8192
_NLC = L // _LC


def _proj_body(w_ref, x_ref, o_ref):
    o_ref[...] = jnp.dot(w_ref[...], x_ref[...],
                         preferred_element_type=jnp.float32)


def _proj(w2d, x2d):
    o, i = w2d.shape
    return pl.pallas_call(
        _proj_body,
        grid=(_NLC,),
        in_specs=[
            pl.BlockSpec((o, i), lambda j: (0, 0)),
            pl.BlockSpec((i, _LC), lambda j: (0, j)),
        ],
        out_specs=pl.BlockSpec((o, _LC), lambda j: (0, j)),
        out_shape=jax.ShapeDtypeStruct((o, L), jnp.float32),
    )(w2d, x2d)


_NROW = DIM
_NSC = 2
_NTILE = 16
_CHUNK = L // _NTILE
_SUB = _CHUNK // 16
_RADIX = 256


def _sc_sort_body(v_hbm, vs_hbm, idx_hbm,
                  fbuf, keybuf, ibuf, posbuf, h2, run2, b2, hgall, histb,
                  baseb, ak, ai, bk, bi, hg, sem1, sem2):
    core = lax.axis_index("c")
    t = lax.axis_index("s")
    lanes = lax.iota(jnp.int32, 16)
    gidx0 = lanes * _SUB
    cbase = t * _CHUNK
    minint = jnp.int32(-2147483648)

    def zero_counts(ref):
        def zb(j, _):
            ref[pl.ds(j * 16, 16)] = jnp.zeros((16,), jnp.int32)
            return 0
        lax.fori_loop(0, _RADIX, zb, 0)

    def sweep_hist(sh):
        zero_counts(h2)

        def body(j, _):
            kv = plsc.load_gather(keybuf, [gidx0 + j])
            d = lax.shift_right_logical(kv, sh) & 255
            cidx = d * 16 + lanes
            h = plsc.load_gather(h2, [cidx])
            plsc.store_scatter(h2, [cidx], h + 1)
            return 0
        lax.fori_loop(0, _SUB, body, 0)
        def tot(g, _):
            dd = lanes + jnp.full((16,), g * 16, jnp.int32)
            def lsum(l, acc):
                return acc + plsc.load_gather(h2, [dd * 16 + l])
            acc = lax.fori_loop(0, 16, lsum, jnp.zeros((16,), jnp.int32))
            histb[pl.ds(g * 16, 16)] = acc
            return 0
        lax.fori_loop(0, _RADIX // 16, tot, 0)
        def expref(d, _):
            row = h2[pl.ds(d * 16, 16)]
            h2[pl.ds(d * 16, 16)] = plsc.cumsum(row) - row
            return 0
        lax.fori_loop(0, _RADIX, expref, 0)

    def combine():
        pltpu.sync_copy(hg, hgall)
        def dig(g, carry):
            acc = jnp.zeros((16,), jnp.int32)
            mine = jnp.zeros((16,), jnp.int32)
            def tt_body(tt, c):
                acc, mine = c
                row = hgall[tt, pl.ds(g * 16, 16)]
                mine = jnp.where(jnp.full((16,), tt, jnp.int32)
                                 < jnp.full((16,), t, jnp.int32),
                                 mine + row, mine)
                return (acc + row, mine)
            acc, mine = lax.fori_loop(0, _NTILE, tt_body, (acc, mine))
            ex = plsc.cumsum(acc) - acc + jnp.full((16,), carry, jnp.int32)
            baseb[pl.ds(g * 16, 16)] = ex + mine
            return carry + jnp.sum(acc, axis=0)
        lax.fori_loop(0, _RADIX // 16, dig, jnp.int32(0))
        def bd(g, _):
            dd = lanes + jnp.full((16,), g * 16, jnp.int32)
            bv = baseb[pl.ds(g * 16, 16)]
            def lb(l, _):
                cidx = dd * 16 + l
                plsc.store_scatter(b2, [cidx],
                                   plsc.load_gather(h2, [cidx]) + bv)
                return 0
            lax.fori_loop(0, 16, lb, 0)
            return 0
        lax.fori_loop(0, _RADIX // 16, bd, 0)

    def sweep_rank(sh):
        zero_counts(run2)

        def body(j, _):
            kv = plsc.load_gather(keybuf, [gidx0 + j])
            d = lax.shift_right_logical(kv, sh) & 255
            cidx = d * 16 + lanes
            c = plsc.load_gather(run2, [cidx])
            b = plsc.load_gather(b2, [cidx])
            plsc.store_scatter(run2, [cidx], c + 1)
            plsc.store_scatter(posbuf, [gidx0 + j], b + c)
            return 0
        lax.fori_loop(0, _SUB, body, 0)

    def one_pass(sh, src_k, src_i, dst_k, dst_i, first, row):
        if first:
            pltpu.sync_copy(v_hbm.at[row, pl.ds(cbase, _CHUNK)], fbuf)
            def keyb(j, _):
                x = fbuf[pl.ds(j * 16, 16)]
                k = lax.bitcast_convert_type(x, jnp.int32)
                m = lax.shift_right_arithmetic(k, 31)
                keybuf[pl.ds(j * 16, 16)] = k ^ (m | minint)
                ibuf[pl.ds(j * 16, 16)] = lanes + jnp.full(
                    (16,), cbase + j * 16, jnp.int32)
                return 0
            lax.fori_loop(0, _SUB, keyb, 0)
        else:
            pltpu.sync_copy(src_k.at[pl.ds(cbase, _CHUNK)], keybuf)
            pltpu.sync_copy(src_i.at[pl.ds(cbase, _CHUNK)], ibuf)
        sweep_hist(sh)
        pltpu.sync_copy(histb, hg.at[t])
        plsc.subcore_barrier()
        combine()
        sweep_rank(sh)
        d1 = pltpu.async_copy(keybuf, dst_k.at[posbuf], sem1)
        d2 = pltpu.async_copy(ibuf, dst_i.at[posbuf], sem2)
        d1.wait()
        d2.wait()
        plsc.subcore_barrier()

    def do_row(r, _):
        row = core * (_NROW // _NSC) + r
        one_pass(jnp.int32(0), None, None, bk, bi, True, row)
        one_pass(jnp.int32(8), bk, bi, ak, ai, False, row)
        one_pass(jnp.int32(16), ak, ai, bk, bi, False, row)
        one_pass(jnp.int32(24), bk, bi, ak, ai, False, row)
        pltpu.sync_copy(ak.at[pl.ds(cbase, _CHUNK)], keybuf)
        pltpu.sync_copy(ai.at[pl.ds(cbase, _CHUNK)], ibuf)
        def unk(j, _):
            kv = keybuf[pl.ds(j * 16, 16)]
            m = lax.shift_right_arithmetic(kv, 31)
            orig = kv ^ jnp.where(m != 0, minint, jnp.int32(-1))
            fbuf[pl.ds(j * 16, 16)] = lax.bitcast_convert_type(orig, jnp.float32)
            return 0
        lax.fori_loop(0, _SUB, unk, 0)
        pltpu.sync_copy(fbuf, vs_hbm.at[row, pl.ds(cbase, _CHUNK)])
        pltpu.sync_copy(ibuf, idx_hbm.at[row, pl.ds(cbase, _CHUNK)])
        plsc.subcore_barrier()
        return 0

    lax.fori_loop(0, _NROW // _NSC, do_row, 0)


def _sc_argsort(v):
    mesh = plsc.VectorSubcoreMesh(core_axis_name="c", subcore_axis_name="s")
    f = pl.kernel(
        _sc_sort_body,
        out_type=(jax.ShapeDtypeStruct((_NROW, L), jnp.float32),
                  jax.ShapeDtypeStruct((_NROW, L), jnp.int32)),
        mesh=mesh,
        compiler_params=pltpu.CompilerParams(needs_layout_passes=False),
        scratch_types=[
            pltpu.VMEM((_CHUNK,), jnp.float32),
            pltpu.VMEM((_CHUNK,), jnp.int32),
            pltpu.VMEM((_CHUNK,), jnp.int32),
            pltpu.VMEM((_CHUNK,), jnp.int32),
            pltpu.VMEM((_RADIX * 16,), jnp.int32),
            pltpu.VMEM((_RADIX * 16,), jnp.int32),
            pltpu.VMEM((_RADIX * 16,), jnp.int32),
            pltpu.VMEM((_NTILE, _RADIX), jnp.int32),
            pltpu.VMEM((_RADIX,), jnp.int32),
            pltpu.VMEM((_RADIX,), jnp.int32),
            pltpu.VMEM_SHARED((L,), jnp.int32),
            pltpu.VMEM_SHARED((L,), jnp.int32),
            pltpu.VMEM_SHARED((L,), jnp.int32),
            pltpu.VMEM_SHARED((L,), jnp.int32),
            pltpu.VMEM_SHARED((_NTILE, _RADIX), jnp.int32),
            pltpu.SemaphoreType.DMA,
            pltpu.SemaphoreType.DMA,
        ])
    return f(v)


def _scatter_axis(idx, vals, axis):
    grids = list(jnp.indices(idx.shape))
    grids[axis] = idx
    return jnp.zeros_like(vals).at[tuple(grids)].set(vals)


def _fold_box(t):
    return t.reshape(HEADS, CPH, HEADS, HW4).reshape(HEADS, DIM, HW4)


def _unfold_box(t):
    return t.reshape(HEADS, CPH, HEADS, HW4).reshape(DIM, L)


def _perm_nb():
    return (jnp.arange(HW4, dtype=jnp.int32)[None, :] * HEADS
            + jnp.arange(HEADS, dtype=jnp.int32)[:, None]).reshape(L)


def _iperm_nb():
    return (jnp.arange(HEADS, dtype=jnp.int32)[None, :] * HW4
            + jnp.arange(HW4, dtype=jnp.int32)[:, None]).reshape(L)


def kernel(x, w_qkv, w_dw, w_out, temperature):
    xs = x[0]
    half = DIM // 2

    xh = xs[:half]
    idx_h = jnp.argsort(xh, axis=-2)
    x_sort = jnp.take_along_axis(xh, idx_h, axis=-2)
    idx_w = jnp.argsort(x_sort, axis=-1)
    x_sort = jnp.take_along_axis(x_sort, idx_w, axis=-1)
    xs = xs.at[:half].set(x_sort)

    _c = _proj(w_qkv[:, :, 0, 0], xs.reshape(DIM, L)).reshape(5 * DIM, H, W)
    _cp = jnp.pad(_c, ((0, 0), (1, 7), (1, 1)))
    qkv = _dw_conv(_cp, w_dw.reshape(5 * DIM, 9))
    q1, k1, q2, k2, v = jnp.split(qkv.reshape(5, DIM, L), 5, axis=0)
    q1, k1, q2, k2, v = q1[0], k1[0], q2[0], k2[0], v[0]

    vs, idx = _sc_argsort(v)
    idx2 = jnp.take(idx, _perm_nb(), axis=1)
    g = lambda t: jnp.take_along_axis(t, idx, axis=-1)
    g2 = lambda t: jnp.take_along_axis(t, idx2, axis=-1)
    q1s, k1s = g(q1), g(k1)
    q2s_nb, k2s_nb, vs_nb = g2(q2), g2(k2), g2(v)

    temp_b = jnp.broadcast_to(temperature.reshape(HEADS, 1, 1), (HEADS, 8, 128))

    qk1 = jnp.concatenate([_fold_box(q1s), _fold_box(k1s)], axis=1)
    attn1 = _attn_softmax(_gram(qk1), temp_b)
    out1 = _mix(attn1, _fold_box(vs))

    qk2 = jnp.concatenate([_fold_box(q2s_nb), _fold_box(k2s_nb)], axis=1)
    attn2 = _attn_softmax(_gram(qk2), temp_b)
    out2 = _mix(attn2, _fold_box(vs_nb))

    out2n = jnp.take(_unfold_box(out2), _iperm_nb(), axis=1)
    prod = _unfold_box(out1) * out2n
    res = _scatter_axis(idx, prod, 1)

    out = _proj(w_out[:, :, 0, 0], res)
    out = out.reshape(DIM, H, W)

    orp = out[:half]
    orp = _scatter_axis(idx_w, orp, 2)
    orp = _scatter_axis(idx_h, orp, 1)
    out = out.at[:half].set(orp)
    return out[None]


# speedup vs baseline: 2.8065x; 1.7041x over previous
import functools

import jax
import jax.numpy as jnp
from jax import lax
from jax.experimental import pallas as pl
from jax.experimental.pallas import tpu as pltpu
from jax.experimental.pallas import tpu_sc as plsc

DIM = 96
HEADS = 4
H = W = 384
L = H * W
HW4 = L // HEADS
CPH = DIM // HEADS


_HB = 8
_NHB = H // _HB


def _qkv_body(xa_ref, xb_ref, w_ref, dw_ref, out_ref):
    x16 = jnp.concatenate([xa_ref[...], xb_ref[...]], axis=1)
    xh = x16[:, 0:_HB + 2, :]
    mm = jnp.dot(w_ref[...], xh.reshape(DIM, -1),
                 preferred_element_type=jnp.float32,
                 precision=lax.Precision.HIGHEST)
    qh = mm.reshape(DIM, _HB + 2, W + 2)
    acc = jnp.zeros((DIM, _HB, W), dtype=jnp.float32)
    for di in range(3):
        for dj in range(3):
            tap = dw_ref[:, 3 * di + dj][:, None, None]
            acc = acc + qh[:, di:di + _HB, dj:dj + W] * tap
    out_ref[...] = acc


def _dw_body(xa_ref, xb_ref, dw_ref, out_ref):
    x16 = jnp.concatenate([xa_ref[...], xb_ref[...]], axis=1)
    qh = x16[:, 0:_HB + 2, :].astype(jnp.bfloat16).astype(jnp.float32)
    dw = dw_ref[...]
    acc = jnp.zeros((DIM, _HB, W), dtype=jnp.float32)
    for di in range(3):
        for dj in range(3):
            tap = dw[:, 3 * di + dj][:, None, None]
            acc = acc + qh[:, di:di + _HB, dj:dj + W] * tap
    out_ref[...] = acc


def _dw_conv(q_pad, dw2d):
    return pl.pallas_call(
        _dw_body,
        grid=(5, _NHB),
        in_specs=[
            pl.BlockSpec((DIM, _HB, W + 2), lambda cb, hb: (cb, hb, 0)),
            pl.BlockSpec((DIM, _HB, W + 2), lambda cb, hb: (cb, hb + 1, 0)),
            pl.BlockSpec((DIM, 9), lambda cb, hb: (cb, 0)),
        ],
        out_specs=pl.BlockSpec((DIM, _HB, W), lambda cb, hb: (cb, hb, 0)),
        out_shape=jax.ShapeDtypeStruct((5 * DIM, H, W), jnp.float32),
    )(q_pad, q_pad, dw2d)


def _qkv_conv(x_pad, w2d, dw2d):
    grid = (5, _NHB)
    return pl.pallas_call(
        _qkv_body,
        grid=grid,
        in_specs=[
            pl.BlockSpec((DIM, _HB, W + 2), lambda cb, hb: (0, hb, 0)),
            pl.BlockSpec((DIM, _HB, W + 2), lambda cb, hb: (0, hb + 1, 0)),
            pl.BlockSpec((DIM, DIM), lambda cb, hb: (cb, 0)),
            pl.BlockSpec((DIM, 9), lambda cb, hb: (cb, 0)),
        ],
        out_specs=pl.BlockSpec((DIM, _HB, W), lambda cb, hb: (cb, hb, 0)),
        out_shape=jax.ShapeDtypeStruct((5 * DIM, H, W), jnp.float32),
    )(x_pad, x_pad, w2d, dw2d)


_PC = 4096
_NPC = HW4 // _PC


def _gram_body(qk_ref, g_ref):
    @pl.when(pl.program_id(1) == 0)
    def _():
        g_ref[...] = jnp.zeros_like(g_ref)
    qk = qk_ref[0]
    g_ref[0] += jnp.dot(qk, qk.T, preferred_element_type=jnp.float32)


def _gram(qk):
    return pl.pallas_call(
        _gram_body,
        grid=(HEADS, _NPC),
        in_specs=[pl.BlockSpec((1, 2 * DIM, _PC), lambda h, p: (h, 0, p))],
        out_specs=pl.BlockSpec((1, 2 * DIM, 2 * DIM), lambda h, p: (h, 0, 0)),
        out_shape=jax.ShapeDtypeStruct((HEADS, 2 * DIM, 2 * DIM), jnp.float32),
    )(qk)


def _attn_body(g_ref, t_ref, a_ref):
    g = g_ref[0]
    n = 2 * DIM
    eye = (lax.broadcasted_iota(jnp.int32, (n, n), 0)
           == lax.broadcasted_iota(jnp.int32, (n, n), 1)).astype(jnp.float32)
    diag = jnp.sum(g * eye, axis=1)
    inv = 1.0 / jnp.maximum(jnp.sqrt(diag), 1e-12)
    sim = g[:DIM, DIM:] * inv[:DIM, None] * inv[None, DIM:]
    t = t_ref[0][0:1, 0:1]
    e = jnp.exp(sim * t)
    a_ref[0] = e / (jnp.sum(e, axis=1, keepdims=True) + 1.0)


def _attn_softmax(g, temp_b):
    return pl.pallas_call(
        _attn_body,
        grid=(HEADS,),
        in_specs=[
            pl.BlockSpec((1, 2 * DIM, 2 * DIM), lambda h: (h, 0, 0)),
            pl.BlockSpec((1, 8, 128), lambda h: (h, 0, 0)),
        ],
        out_specs=pl.BlockSpec((1, DIM, DIM), lambda h: (h, 0, 0)),
        out_shape=jax.ShapeDtypeStruct((HEADS, DIM, DIM), jnp.float32),
    )(g, temp_b)


def _mix_body(a_ref, v_ref, o_ref):
    o_ref[0] = jnp.dot(a_ref[0], v_ref[0], preferred_element_type=jnp.float32)


def _mix(attn, v):
    return pl.pallas_call(
        _mix_body,
        grid=(HEADS, _NPC),
        in_specs=[
            pl.BlockSpec((1, DIM, DIM), lambda h, p: (h, 0, 0)),
            pl.BlockSpec((1, DIM, _PC), lambda h, p: (h, 0, p)),
        ],
        out_specs=pl.BlockSpec((1, DIM, _PC), lambda h, p: (h, 0, p)),
        out_shape=jax.ShapeDtypeStruct((HEADS, DIM, HW4), jnp.float32),
    )(attn, v)


_LC = 8192
_NLC = L // _LC


def _proj_body(w_ref, x_ref, o_ref):
    o_ref[...] = jnp.dot(w_ref[...], x_ref[...],
                         preferred_element_type=jnp.float32)


def _proj(w2d, x2d):
    o, i = w2d.shape
    return pl.pallas_call(
        _proj_body,
        grid=(_NLC,),
        in_specs=[
            pl.BlockSpec((o, i), lambda j: (0, 0)),
            pl.BlockSpec((i, _LC), lambda j: (0, j)),
        ],
        out_specs=pl.BlockSpec((o, _LC), lambda j: (0, j)),
        out_shape=jax.ShapeDtypeStruct((o, L), jnp.float32),
    )(w2d, x2d)


_NROW = DIM
_NSC = 2
_NTILE = 16
_CHUNK = L // _NTILE
_SUB = _CHUNK // 16
_RADIX = 256


def _sc_sort_body(v_hbm, vs_hbm, idx_hbm, rank_hbm,
                  fbuf, keybuf, ibuf, posbuf, h2, run2, b2, hgall, histb,
                  baseb, ak, ai, bk, bi, rs, hg, sem1, sem2, sem3):
    core = lax.axis_index("c")
    t = lax.axis_index("s")
    lanes = lax.iota(jnp.int32, 16)
    gidx0 = lanes * _SUB
    cbase = t * _CHUNK
    minint = jnp.int32(-2147483648)

    def zero_counts(ref):
        def zb(j, _):
            ref[pl.ds(j * 16, 16)] = jnp.zeros((16,), jnp.int32)
            return 0
        lax.fori_loop(0, _RADIX, zb, 0)

    def sweep_hist(sh):
        zero_counts(h2)

        def body(j, _):
            kv = plsc.load_gather(keybuf, [gidx0 + j])
            d = lax.shift_right_logical(kv, sh) & 255
            cidx = d * 16 + lanes
            h = plsc.load_gather(h2, [cidx])
            plsc.store_scatter(h2, [cidx], h + 1)
            return 0
        lax.fori_loop(0, _SUB, body, 0)
        def tot(g, _):
            dd = lanes + jnp.full((16,), g * 16, jnp.int32)
            def lsum(l, acc):
                return acc + plsc.load_gather(h2, [dd * 16 + l])
            acc = lax.fori_loop(0, 16, lsum, jnp.zeros((16,), jnp.int32))
            histb[pl.ds(g * 16, 16)] = acc
            return 0
        lax.fori_loop(0, _RADIX // 16, tot, 0)
        def expref(d, _):
            row = h2[pl.ds(d * 16, 16)]
            h2[pl.ds(d * 16, 16)] = plsc.cumsum(row) - row
            return 0
        lax.fori_loop(0, _RADIX, expref, 0)

    def combine():
        pltpu.sync_copy(hg, hgall)
        def dig(g, carry):
            acc = jnp.zeros((16,), jnp.int32)
            mine = jnp.zeros((16,), jnp.int32)
            def tt_body(tt, c):
                acc, mine = c
                row = hgall[tt, pl.ds(g * 16, 16)]
                mine = jnp.where(jnp.full((16,), tt, jnp.int32)
                                 < jnp.full((16,), t, jnp.int32),
                                 mine + row, mine)
                return (acc + row, mine)
            acc, mine = lax.fori_loop(0, _NTILE, tt_body, (acc, mine))
            ex = plsc.cumsum(acc) - acc + jnp.full((16,), carry, jnp.int32)
            baseb[pl.ds(g * 16, 16)] = ex + mine
            return carry + jnp.sum(acc, axis=0)
        lax.fori_loop(0, _RADIX // 16, dig, jnp.int32(0))
        def bd(g, _):
            dd = lanes + jnp.full((16,), g * 16, jnp.int32)
            bv = baseb[pl.ds(g * 16, 16)]
            def lb(l, _):
                cidx = dd * 16 + l
                plsc.store_scatter(b2, [cidx],
                                   plsc.load_gather(h2, [cidx]) + bv)
                return 0
            lax.fori_loop(0, 16, lb, 0)
            return 0
        lax.fori_loop(0, _RADIX // 16, bd, 0)

    def sweep_rank(sh):
        zero_counts(run2)

        def body(j, _):
            kv = plsc.load_gather(keybuf, [gidx0 + j])
            d = lax.shift_right_logical(kv, sh) & 255
            cidx = d * 16 + lanes
            c = plsc.load_gather(run2, [cidx])
            b = plsc.load_gather(b2, [cidx])
            plsc.store_scatter(run2, [cidx], c + 1)
            plsc.store_scatter(posbuf, [gidx0 + j], b + c)
            return 0
        lax.fori_loop(0, _SUB, body, 0)

    def one_pass(sh, src_k, src_i, dst_k, dst_i, first, row, last=False):
        if first:
            pltpu.sync_copy(v_hbm.at[row, pl.ds(cbase, _CHUNK)], fbuf)
            def keyb(j, _):
                x = fbuf[pl.ds(j * 16, 16)]
                k = lax.bitcast_convert_type(x, jnp.int32)
                m = lax.shift_right_arithmetic(k, 31)
                keybuf[pl.ds(j * 16, 16)] = k ^ (m | minint)
                ibuf[pl.ds(j * 16, 16)] = lanes + jnp.full(
                    (16,), cbase + j * 16, jnp.int32)
                return 0
            lax.fori_loop(0, _SUB, keyb, 0)
        else:
            pltpu.sync_copy(src_k.at[pl.ds(cbase, _CHUNK)], keybuf)
            pltpu.sync_copy(src_i.at[pl.ds(cbase, _CHUNK)], ibuf)
        sweep_hist(sh)
        pltpu.sync_copy(histb, hg.at[t])
        plsc.subcore_barrier()
        combine()
        sweep_rank(sh)
        d1 = pltpu.async_copy(keybuf, dst_k.at[posbuf], sem1)
        d2 = pltpu.async_copy(ibuf, dst_i.at[posbuf], sem2)
        if last:
            d3 = pltpu.async_copy(posbuf, rs.at[ibuf], sem3)
            d3.wait()
        d1.wait()
        d2.wait()
        plsc.subcore_barrier()

    def do_row(r, _):
        row = core * (_NROW // _NSC) + r
        one_pass(jnp.int32(0), None, None, bk, bi, True, row)
        one_pass(jnp.int32(8), bk, bi, ak, ai, False, row)
        one_pass(jnp.int32(16), ak, ai, bk, bi, False, row)
        one_pass(jnp.int32(24), bk, bi, ak, ai, False, row, last=True)
        pltpu.sync_copy(ak.at[pl.ds(cbase, _CHUNK)], keybuf)
        pltpu.sync_copy(ai.at[pl.ds(cbase, _CHUNK)], ibuf)
        def unk(j, _):
            kv = keybuf[pl.ds(j * 16, 16)]
            m = lax.shift_right_arithmetic(kv, 31)
            orig = kv ^ jnp.where(m != 0, minint, jnp.int32(-1))
            fbuf[pl.ds(j * 16, 16)] = lax.bitcast_convert_type(orig, jnp.float32)
            return 0
        lax.fori_loop(0, _SUB, unk, 0)
        pltpu.sync_copy(fbuf, vs_hbm.at[row, pl.ds(cbase, _CHUNK)])
        pltpu.sync_copy(ibuf, idx_hbm.at[row, pl.ds(cbase, _CHUNK)])
        pltpu.sync_copy(rs.at[pl.ds(cbase, _CHUNK)], keybuf)
        pltpu.sync_copy(keybuf, rank_hbm.at[row, pl.ds(cbase, _CHUNK)])
        plsc.subcore_barrier()
        return 0

    lax.fori_loop(0, _NROW // _NSC, do_row, 0)


def _sc_argsort(v):
    mesh = plsc.VectorSubcoreMesh(core_axis_name="c", subcore_axis_name="s")
    f = pl.kernel(
        _sc_sort_body,
        out_type=(jax.ShapeDtypeStruct((_NROW, L), jnp.float32),
                  jax.ShapeDtypeStruct((_NROW, L), jnp.int32),
                  jax.ShapeDtypeStruct((_NROW, L), jnp.int32)),
        mesh=mesh,
        compiler_params=pltpu.CompilerParams(needs_layout_passes=False),
        scratch_types=[
            pltpu.VMEM((_CHUNK,), jnp.float32),
            pltpu.VMEM((_CHUNK,), jnp.int32),
            pltpu.VMEM((_CHUNK,), jnp.int32),
            pltpu.VMEM((_CHUNK,), jnp.int32),
            pltpu.VMEM((_RADIX * 16,), jnp.int32),
            pltpu.VMEM((_RADIX * 16,), jnp.int32),
            pltpu.VMEM((_RADIX * 16,), jnp.int32),
            pltpu.VMEM((_NTILE, _RADIX), jnp.int32),
            pltpu.VMEM((_RADIX,), jnp.int32),
            pltpu.VMEM((_RADIX,), jnp.int32),
            pltpu.VMEM_SHARED((L,), jnp.int32),
            pltpu.VMEM_SHARED((L,), jnp.int32),
            pltpu.VMEM_SHARED((L,), jnp.int32),
            pltpu.VMEM_SHARED((L,), jnp.int32),
            pltpu.VMEM_SHARED((L,), jnp.int32),
            pltpu.VMEM_SHARED((_NTILE, _RADIX), jnp.int32),
            pltpu.SemaphoreType.DMA,
            pltpu.SemaphoreType.DMA,
            pltpu.SemaphoreType.DMA,
        ])
    return f(v)


def _scatter_axis(idx, vals, axis):
    grids = list(jnp.indices(idx.shape))
    grids[axis] = idx
    return jnp.zeros_like(vals).at[tuple(grids)].set(vals)


def _fold_box(t):
    return t.reshape(HEADS, CPH, HEADS, HW4).reshape(HEADS, DIM, HW4)


def _unfold_box(t):
    return t.reshape(HEADS, CPH, HEADS, HW4).reshape(DIM, L)


def _perm_nb():
    return (jnp.arange(HW4, dtype=jnp.int32)[None, :] * HEADS
            + jnp.arange(HEADS, dtype=jnp.int32)[:, None]).reshape(L)


def _iperm_nb():
    return (jnp.arange(HEADS, dtype=jnp.int32)[None, :] * HW4
            + jnp.arange(HW4, dtype=jnp.int32)[:, None]).reshape(L)


def kernel(x, w_qkv, w_dw, w_out, temperature):
    xs = x[0]
    half = DIM // 2

    xh = xs[:half]
    idx_h = jnp.argsort(xh, axis=-2)
    x_sort = jnp.take_along_axis(xh, idx_h, axis=-2)
    idx_w = jnp.argsort(x_sort, axis=-1)
    x_sort = jnp.take_along_axis(x_sort, idx_w, axis=-1)
    xs = xs.at[:half].set(x_sort)

    _c = _proj(w_qkv[:, :, 0, 0], xs.reshape(DIM, L)).reshape(5 * DIM, H, W)
    _cp = jnp.pad(_c, ((0, 0), (1, 7), (1, 1)))
    qkv = _dw_conv(_cp, w_dw.reshape(5 * DIM, 9))
    q1, k1, q2, k2, v = jnp.split(qkv.reshape(5, DIM, L), 5, axis=0)
    q1, k1, q2, k2, v = q1[0], k1[0], q2[0], k2[0], v[0]

    vs, idx, rank = _sc_argsort(v)
    idx2 = jnp.take(idx, _perm_nb(), axis=1)
    g = lambda t: jnp.take_along_axis(t, idx, axis=-1)
    g2 = lambda t: jnp.take_along_axis(t, idx2, axis=-1)
    q1s, k1s = g(q1), g(k1)
    q2s_nb, k2s_nb, vs_nb = g2(q2), g2(k2), g2(v)

    temp_b = jnp.broadcast_to(temperature.reshape(HEADS, 1, 1), (HEADS, 8, 128))

    qk1 = jnp.concatenate([_fold_box(q1s), _fold_box(k1s)], axis=1)
    attn1 = _attn_softmax(_gram(qk1), temp_b)
    out1 = _mix(attn1, _fold_box(vs))

    qk2 = jnp.concatenate([_fold_box(q2s_nb), _fold_box(k2s_nb)], axis=1)
    attn2 = _attn_softmax(_gram(qk2), temp_b)
    out2 = _mix(attn2, _fold_box(vs_nb))

    out2n = jnp.take(_unfold_box(out2), _iperm_nb(), axis=1)
    prod = _unfold_box(out1) * out2n
    res = jnp.take_along_axis(prod, rank, axis=-1)

    out = _proj(w_out[:, :, 0, 0], res)
    out = out.reshape(DIM, H, W)

    orp = out[:half]
    orp = _scatter_axis(idx_w, orp, 2)
    orp = _scatter_axis(idx_h, orp, 1)
    out = out.at[:half].set(orp)
    return out[None]


# speedup vs baseline: 9.6004x; 3.4207x over previous
import functools

import jax
import jax.numpy as jnp
from jax import lax
from jax.experimental import pallas as pl
from jax.experimental.pallas import tpu as pltpu
from jax.experimental.pallas import tpu_sc as plsc

DIM = 96
HEADS = 4
H = W = 384
L = H * W
HW4 = L // HEADS
CPH = DIM // HEADS


_HB = 8
_NHB = H // _HB


def _qkv_body(xa_ref, xb_ref, w_ref, dw_ref, out_ref):
    x16 = jnp.concatenate([xa_ref[...], xb_ref[...]], axis=1)
    xh = x16[:, 0:_HB + 2, :]
    mm = jnp.dot(w_ref[...], xh.reshape(DIM, -1),
                 preferred_element_type=jnp.float32,
                 precision=lax.Precision.HIGHEST)
    qh = mm.reshape(DIM, _HB + 2, W + 2)
    acc = jnp.zeros((DIM, _HB, W), dtype=jnp.float32)
    for di in range(3):
        for dj in range(3):
            tap = dw_ref[:, 3 * di + dj][:, None, None]
            acc = acc + qh[:, di:di + _HB, dj:dj + W] * tap
    out_ref[...] = acc


def _dw_body(xa_ref, xb_ref, dw_ref, out_ref):
    x16 = jnp.concatenate([xa_ref[...], xb_ref[...]], axis=1)
    qh = x16[:, 0:_HB + 2, :].astype(jnp.bfloat16).astype(jnp.float32)
    dw = dw_ref[...]
    acc = jnp.zeros((DIM, _HB, W), dtype=jnp.float32)
    for di in range(3):
        for dj in range(3):
            tap = dw[:, 3 * di + dj][:, None, None]
            acc = acc + qh[:, di:di + _HB, dj:dj + W] * tap
    out_ref[...] = acc


def _dw_conv(q_pad, dw2d):
    return pl.pallas_call(
        _dw_body,
        grid=(5, _NHB),
        in_specs=[
            pl.BlockSpec((DIM, _HB, W + 2), lambda cb, hb: (cb, hb, 0)),
            pl.BlockSpec((DIM, _HB, W + 2), lambda cb, hb: (cb, hb + 1, 0)),
            pl.BlockSpec((DIM, 9), lambda cb, hb: (cb, 0)),
        ],
        out_specs=pl.BlockSpec((DIM, _HB, W), lambda cb, hb: (cb, hb, 0)),
        out_shape=jax.ShapeDtypeStruct((5 * DIM, H, W), jnp.float32),
    )(q_pad, q_pad, dw2d)


def _qkv_conv(x_pad, w2d, dw2d):
    grid = (5, _NHB)
    return pl.pallas_call(
        _qkv_body,
        grid=grid,
        in_specs=[
            pl.BlockSpec((DIM, _HB, W + 2), lambda cb, hb: (0, hb, 0)),
            pl.BlockSpec((DIM, _HB, W + 2), lambda cb, hb: (0, hb + 1, 0)),
            pl.BlockSpec((DIM, DIM), lambda cb, hb: (cb, 0)),
            pl.BlockSpec((DIM, 9), lambda cb, hb: (cb, 0)),
        ],
        out_specs=pl.BlockSpec((DIM, _HB, W), lambda cb, hb: (cb, hb, 0)),
        out_shape=jax.ShapeDtypeStruct((5 * DIM, H, W), jnp.float32),
    )(x_pad, x_pad, w2d, dw2d)


_PC = 4096
_NPC = HW4 // _PC


def _gram_body(qk_ref, g_ref):
    @pl.when(pl.program_id(1) == 0)
    def _():
        g_ref[...] = jnp.zeros_like(g_ref)
    qk = qk_ref[0]
    g_ref[0] += jnp.dot(qk, qk.T, preferred_element_type=jnp.float32)


def _gram(qk):
    return pl.pallas_call(
        _gram_body,
        grid=(HEADS, _NPC),
        in_specs=[pl.BlockSpec((1, 2 * DIM, _PC), lambda h, p: (h, 0, p))],
        out_specs=pl.BlockSpec((1, 2 * DIM, 2 * DIM), lambda h, p: (h, 0, 0)),
        out_shape=jax.ShapeDtypeStruct((HEADS, 2 * DIM, 2 * DIM), jnp.float32),
    )(qk)


def _attn_body(g_ref, t_ref, a_ref):
    g = g_ref[0]
    n = 2 * DIM
    eye = (lax.broadcasted_iota(jnp.int32, (n, n), 0)
           == lax.broadcasted_iota(jnp.int32, (n, n), 1)).astype(jnp.float32)
    diag = jnp.sum(g * eye, axis=1)
    inv = 1.0 / jnp.maximum(jnp.sqrt(diag), 1e-12)
    sim = g[:DIM, DIM:] * inv[:DIM, None] * inv[None, DIM:]
    t = t_ref[0][0:1, 0:1]
    e = jnp.exp(sim * t)
    a_ref[0] = e / (jnp.sum(e, axis=1, keepdims=True) + 1.0)


def _attn_softmax(g, temp_b):
    return pl.pallas_call(
        _attn_body,
        grid=(HEADS,),
        in_specs=[
            pl.BlockSpec((1, 2 * DIM, 2 * DIM), lambda h: (h, 0, 0)),
            pl.BlockSpec((1, 8, 128), lambda h: (h, 0, 0)),
        ],
        out_specs=pl.BlockSpec((1, DIM, DIM), lambda h: (h, 0, 0)),
        out_shape=jax.ShapeDtypeStruct((HEADS, DIM, DIM), jnp.float32),
    )(g, temp_b)


def _mix_body(a_ref, v_ref, o_ref):
    o_ref[0] = jnp.dot(a_ref[0], v_ref[0], preferred_element_type=jnp.float32)


def _mix(attn, v):
    return pl.pallas_call(
        _mix_body,
        grid=(HEADS, _NPC),
        in_specs=[
            pl.BlockSpec((1, DIM, DIM), lambda h, p: (h, 0, 0)),
            pl.BlockSpec((1, DIM, _PC), lambda h, p: (h, 0, p)),
        ],
        out_specs=pl.BlockSpec((1, DIM, _PC), lambda h, p: (h, 0, p)),
        out_shape=jax.ShapeDtypeStruct((HEADS, DIM, HW4), jnp.float32),
    )(attn, v)


_LC = 8192
_NLC = L // _LC


def _proj_body(w_ref, x_ref, o_ref):
    o_ref[...] = jnp.dot(w_ref[...], x_ref[...],
                         preferred_element_type=jnp.float32)


def _proj(w2d, x2d):
    o, i = w2d.shape
    return pl.pallas_call(
        _proj_body,
        grid=(_NLC,),
        in_specs=[
            pl.BlockSpec((o, i), lambda j: (0, 0)),
            pl.BlockSpec((i, _LC), lambda j: (0, j)),
        ],
        out_specs=pl.BlockSpec((o, _LC), lambda j: (0, j)),
        out_shape=jax.ShapeDtypeStruct((o, L), jnp.float32),
    )(w2d, x2d)


_NROW = DIM
_NSC = 2
_NTILE = 16
_CHUNK = L // _NTILE
_SUB = _CHUNK // 16
_RADIX = 256


def _sc_sort_body(v_hbm, vs_hbm, idx_hbm, rank_hbm,
                  fbuf, keybuf, ibuf, posbuf, h2, run2, b2, hgall, histb,
                  baseb, ak, ai, bk, bi, rs, hg, sem1, sem2, sem3):
    core = lax.axis_index("c")
    t = lax.axis_index("s")
    lanes = lax.iota(jnp.int32, 16)
    gidx0 = lanes * _SUB
    cbase = t * _CHUNK
    minint = jnp.int32(-2147483648)

    def zero_counts(ref):
        def zb(j, _):
            ref[pl.ds(j * 16, 16)] = jnp.zeros((16,), jnp.int32)
            return 0
        lax.fori_loop(0, _RADIX, zb, 0)

    def sweep_hist(sh):
        zero_counts(h2)

        def body(j, _):
            kv = plsc.load_gather(keybuf, [gidx0 + j])
            d = lax.shift_right_logical(kv, sh) & 255
            cidx = d * 16 + lanes
            h = plsc.load_gather(h2, [cidx])
            plsc.store_scatter(h2, [cidx], h + 1)
            return 0
        lax.fori_loop(0, _SUB, body, 0)
        def tot(g, _):
            dd = lanes + jnp.full((16,), g * 16, jnp.int32)
            def lsum(l, acc):
                return acc + plsc.load_gather(h2, [dd * 16 + l])
            acc = lax.fori_loop(0, 16, lsum, jnp.zeros((16,), jnp.int32))
            histb[pl.ds(g * 16, 16)] = acc
            return 0
        lax.fori_loop(0, _RADIX // 16, tot, 0)
        def expref(d, _):
            row = h2[pl.ds(d * 16, 16)]
            h2[pl.ds(d * 16, 16)] = plsc.cumsum(row) - row
            return 0
        lax.fori_loop(0, _RADIX, expref, 0)

    def combine():
        pltpu.sync_copy(hg, hgall)
        def dig(g, carry):
            acc = jnp.zeros((16,), jnp.int32)
            mine = jnp.zeros((16,), jnp.int32)
            def tt_body(tt, c):
                acc, mine = c
                row = hgall[tt, pl.ds(g * 16, 16)]
                mine = jnp.where(jnp.full((16,), tt, jnp.int32)
                                 < jnp.full((16,), t, jnp.int32),
                                 mine + row, mine)
                return (acc + row, mine)
            acc, mine = lax.fori_loop(0, _NTILE, tt_body, (acc, mine))
            ex = plsc.cumsum(acc) - acc + jnp.full((16,), carry, jnp.int32)
            baseb[pl.ds(g * 16, 16)] = ex + mine
            return carry + jnp.sum(acc, axis=0)
        lax.fori_loop(0, _RADIX // 16, dig, jnp.int32(0))
        def bd(g, _):
            dd = lanes + jnp.full((16,), g * 16, jnp.int32)
            bv = baseb[pl.ds(g * 16, 16)]
            def lb(l, _):
                cidx = dd * 16 + l
                plsc.store_scatter(b2, [cidx],
                                   plsc.load_gather(h2, [cidx]) + bv)
                return 0
            lax.fori_loop(0, 16, lb, 0)
            return 0
        lax.fori_loop(0, _RADIX // 16, bd, 0)

    def sweep_rank(sh):
        zero_counts(run2)

        def body(j, _):
            kv = plsc.load_gather(keybuf, [gidx0 + j])
            d = lax.shift_right_logical(kv, sh) & 255
            cidx = d * 16 + lanes
            c = plsc.load_gather(run2, [cidx])
            b = plsc.load_gather(b2, [cidx])
            plsc.store_scatter(run2, [cidx], c + 1)
            plsc.store_scatter(posbuf, [gidx0 + j], b + c)
            return 0
        lax.fori_loop(0, _SUB, body, 0)

    def one_pass(sh, src_k, src_i, dst_k, dst_i, first, row, last=False):
        if first:
            pltpu.sync_copy(v_hbm.at[row, pl.ds(cbase, _CHUNK)], fbuf)
            def keyb(j, _):
                x = fbuf[pl.ds(j * 16, 16)]
                k = lax.bitcast_convert_type(x, jnp.int32)
                m = lax.shift_right_arithmetic(k, 31)
                keybuf[pl.ds(j * 16, 16)] = k ^ (m | minint)
                ibuf[pl.ds(j * 16, 16)] = lanes + jnp.full(
                    (16,), cbase + j * 16, jnp.int32)
                return 0
            lax.fori_loop(0, _SUB, keyb, 0)
        else:
            pltpu.sync_copy(src_k.at[pl.ds(cbase, _CHUNK)], keybuf)
            pltpu.sync_copy(src_i.at[pl.ds(cbase, _CHUNK)], ibuf)
        sweep_hist(sh)
        pltpu.sync_copy(histb, hg.at[t])
        plsc.subcore_barrier()
        combine()
        sweep_rank(sh)
        d1 = pltpu.async_copy(keybuf, dst_k.at[posbuf], sem1)
        d2 = pltpu.async_copy(ibuf, dst_i.at[posbuf], sem2)
        if last:
            d3 = pltpu.async_copy(posbuf, rs.at[ibuf], sem3)
            d3.wait()
        d1.wait()
        d2.wait()
        plsc.subcore_barrier()

    def do_row(r, _):
        row = core * (_NROW // _NSC) + r
        one_pass(jnp.int32(0), None, None, bk, bi, True, row)
        one_pass(jnp.int32(8), bk, bi, ak, ai, False, row)
        one_pass(jnp.int32(16), ak, ai, bk, bi, False, row)
        one_pass(jnp.int32(24), bk, bi, ak, ai, False, row, last=True)
        pltpu.sync_copy(ak.at[pl.ds(cbase, _CHUNK)], keybuf)
        pltpu.sync_copy(ai.at[pl.ds(cbase, _CHUNK)], ibuf)
        def unk(j, _):
            kv = keybuf[pl.ds(j * 16, 16)]
            m = lax.shift_right_arithmetic(kv, 31)
            orig = kv ^ jnp.where(m != 0, minint, jnp.int32(-1))
            fbuf[pl.ds(j * 16, 16)] = lax.bitcast_convert_type(orig, jnp.float32)
            return 0
        lax.fori_loop(0, _SUB, unk, 0)
        pltpu.sync_copy(fbuf, vs_hbm.at[row, pl.ds(cbase, _CHUNK)])
        pltpu.sync_copy(ibuf, idx_hbm.at[row, pl.ds(cbase, _CHUNK)])
        pltpu.sync_copy(rs.at[pl.ds(cbase, _CHUNK)], keybuf)
        pltpu.sync_copy(keybuf, rank_hbm.at[row, pl.ds(cbase, _CHUNK)])
        plsc.subcore_barrier()
        return 0

    lax.fori_loop(0, _NROW // _NSC, do_row, 0)


def _sc_argsort(v):
    mesh = plsc.VectorSubcoreMesh(core_axis_name="c", subcore_axis_name="s")
    f = pl.kernel(
        _sc_sort_body,
        out_type=(jax.ShapeDtypeStruct((_NROW, L), jnp.float32),
                  jax.ShapeDtypeStruct((_NROW, L), jnp.int32),
                  jax.ShapeDtypeStruct((_NROW, L), jnp.int32)),
        mesh=mesh,
        compiler_params=pltpu.CompilerParams(needs_layout_passes=False),
        scratch_types=[
            pltpu.VMEM((_CHUNK,), jnp.float32),
            pltpu.VMEM((_CHUNK,), jnp.int32),
            pltpu.VMEM((_CHUNK,), jnp.int32),
            pltpu.VMEM((_CHUNK,), jnp.int32),
            pltpu.VMEM((_RADIX * 16,), jnp.int32),
            pltpu.VMEM((_RADIX * 16,), jnp.int32),
            pltpu.VMEM((_RADIX * 16,), jnp.int32),
            pltpu.VMEM((_NTILE, _RADIX), jnp.int32),
            pltpu.VMEM((_RADIX,), jnp.int32),
            pltpu.VMEM((_RADIX,), jnp.int32),
            pltpu.VMEM_SHARED((L,), jnp.int32),
            pltpu.VMEM_SHARED((L,), jnp.int32),
            pltpu.VMEM_SHARED((L,), jnp.int32),
            pltpu.VMEM_SHARED((L,), jnp.int32),
            pltpu.VMEM_SHARED((L,), jnp.int32),
            pltpu.VMEM_SHARED((_NTILE, _RADIX), jnp.int32),
            pltpu.SemaphoreType.DMA,
            pltpu.SemaphoreType.DMA,
            pltpu.SemaphoreType.DMA,
        ])
    return f(v)


_HC = DIM // 2


def _sc_inv_body(iw_hbm, ih_hbm, comp_hbm, ibuf, dbuf, sbuf, obuf,
                 slw, slh, sem):
    core = lax.axis_index("c")
    t = lax.axis_index("s")
    lanes = lax.iota(jnp.int32, 16)

    def do_c(cc, _):
        c = core * (_HC // _NSC) + cc
        pltpu.sync_copy(iw_hbm.at[c, pl.ds(t * 24, 24), :], ibuf)

        def rw(r, _):
            hrow = t * 24 + r
            def kk(k, _):
                iv = ibuf[r, pl.ds(k * 16, 16)]
                off = (r * 24 + k) * 16
                dbuf[pl.ds(off, 16)] = iv + jnp.full((16,), hrow * W,
                                                     jnp.int32)
                sbuf[pl.ds(off, 16)] = lanes + jnp.full((16,), k * 16,
                                                        jnp.int32)
                return 0
            lax.fori_loop(0, 24, kk, 0)
            return 0
        lax.fori_loop(0, 24, rw, 0)
        pltpu.async_copy(sbuf, slw.at[dbuf], sem).wait()
        pltpu.sync_copy(ih_hbm.at[c, pl.ds(t * 24, 24), :], ibuf)

        def rh(r, _):
            hrow = t * 24 + r
            def kk(k, _):
                iv = ibuf[r, pl.ds(k * 16, 16)]
                off = (r * 24 + k) * 16
                dbuf[pl.ds(off, 16)] = iv * W + lanes + jnp.full(
                    (16,), k * 16, jnp.int32)
                sbuf[pl.ds(off, 16)] = jnp.full((16,), hrow, jnp.int32)
                return 0
            lax.fori_loop(0, 24, kk, 0)
            return 0
        lax.fori_loop(0, 24, rh, 0)
        pltpu.async_copy(sbuf, slh.at[dbuf], sem).wait()
        plsc.subcore_barrier()
        pltpu.sync_copy(slh.at[pl.ds(t * _CHUNK, _CHUNK)], obuf)

        def rc(r, _):
            def kk(k, _):
                off = (r * 24 + k) * 16
                rhv = obuf[pl.ds(off, 16)]
                dbuf[pl.ds(off, 16)] = rhv * W + lanes + jnp.full(
                    (16,), k * 16, jnp.int32)
                return 0
            lax.fori_loop(0, 24, kk, 0)
            return 0
        lax.fori_loop(0, 24, rc, 0)
        pltpu.async_copy(slw.at[dbuf], sbuf, sem).wait()

        def rc2(r, _):
            def kk(k, _):
                off = (r * 24 + k) * 16
                sbuf[pl.ds(off, 16)] = obuf[pl.ds(off, 16)] * W + sbuf[
                    pl.ds(off, 16)]
                return 0
            lax.fori_loop(0, 24, kk, 0)
            return 0
        lax.fori_loop(0, 24, rc2, 0)
        pltpu.sync_copy(sbuf, comp_hbm.at[c, pl.ds(t * _CHUNK, _CHUNK)])
        plsc.subcore_barrier()
        return 0

    lax.fori_loop(0, _HC // _NSC, do_c, 0)


def _sc_invert(idx_w, idx_h):
    mesh = plsc.VectorSubcoreMesh(core_axis_name="c", subcore_axis_name="s")
    f = pl.kernel(
        _sc_inv_body,
        out_type=jax.ShapeDtypeStruct((_HC, L), jnp.int32),
        mesh=mesh,
        compiler_params=pltpu.CompilerParams(needs_layout_passes=False),
        scratch_types=[
            pltpu.VMEM((24, W), jnp.int32),
            pltpu.VMEM((_CHUNK,), jnp.int32),
            pltpu.VMEM((_CHUNK,), jnp.int32),
            pltpu.VMEM((_CHUNK,), jnp.int32),
            pltpu.VMEM_SHARED((L,), jnp.int32),
            pltpu.VMEM_SHARED((L,), jnp.int32),
            pltpu.SemaphoreType.DMA,
        ])
    return f(idx_w, idx_h)


def _scatter_axis(idx, vals, axis):
    grids = list(jnp.indices(idx.shape))
    grids[axis] = idx
    return jnp.zeros_like(vals).at[tuple(grids)].set(vals)


def _fold_box(t):
    return t.reshape(HEADS, CPH, HEADS, HW4).reshape(HEADS, DIM, HW4)


def _unfold_box(t):
    return t.reshape(HEADS, CPH, HEADS, HW4).reshape(DIM, L)


def _perm_nb():
    return (jnp.arange(HW4, dtype=jnp.int32)[None, :] * HEADS
            + jnp.arange(HEADS, dtype=jnp.int32)[:, None]).reshape(L)


def _iperm_nb():
    return (jnp.arange(HEADS, dtype=jnp.int32)[None, :] * HW4
            + jnp.arange(HW4, dtype=jnp.int32)[:, None]).reshape(L)


def kernel(x, w_qkv, w_dw, w_out, temperature):
    xs = x[0]
    half = DIM // 2

    xh = xs[:half]
    idx_h = jnp.argsort(xh, axis=-2)
    x_sort = jnp.take_along_axis(xh, idx_h, axis=-2)
    idx_w = jnp.argsort(x_sort, axis=-1)
    x_sort = jnp.take_along_axis(x_sort, idx_w, axis=-1)
    xs = xs.at[:half].set(x_sort)

    _c = _proj(w_qkv[:, :, 0, 0], xs.reshape(DIM, L)).reshape(5 * DIM, H, W)
    _cp = jnp.pad(_c, ((0, 0), (1, 7), (1, 1)))
    qkv = _dw_conv(_cp, w_dw.reshape(5 * DIM, 9))
    q1, k1, q2, k2, v = jnp.split(qkv.reshape(5, DIM, L), 5, axis=0)
    q1, k1, q2, k2, v = q1[0], k1[0], q2[0], k2[0], v[0]

    vs, idx, rank = _sc_argsort(v)
    idx2 = jnp.take(idx, _perm_nb(), axis=1)
    g = lambda t: jnp.take_along_axis(t, idx, axis=-1)
    g2 = lambda t: jnp.take_along_axis(t, idx2, axis=-1)
    q1s, k1s = g(q1), g(k1)
    q2s_nb, k2s_nb, vs_nb = g2(q2), g2(k2), g2(v)

    temp_b = jnp.broadcast_to(temperature.reshape(HEADS, 1, 1), (HEADS, 8, 128))

    qk1 = jnp.concatenate([_fold_box(q1s), _fold_box(k1s)], axis=1)
    attn1 = _attn_softmax(_gram(qk1), temp_b)
    out1 = _mix(attn1, _fold_box(vs))

    qk2 = jnp.concatenate([_fold_box(q2s_nb), _fold_box(k2s_nb)], axis=1)
    attn2 = _attn_softmax(_gram(qk2), temp_b)
    out2 = _mix(attn2, _fold_box(vs_nb))

    out2n = jnp.take(_unfold_box(out2), _iperm_nb(), axis=1)
    prod = _unfold_box(out1) * out2n
    res = jnp.take_along_axis(prod, rank, axis=-1)

    out = _proj(w_out[:, :, 0, 0], res)

    comp = _sc_invert(idx_w.astype(jnp.int32), idx_h.astype(jnp.int32))
    first = jnp.take_along_axis(out[:half], comp, axis=-1)
    out = jnp.concatenate([first, out[half:]], axis=0)
    return out.reshape(DIM, H, W)[None]


# speedup vs baseline: 9.8784x; 1.0290x over previous
import functools

import jax
import jax.numpy as jnp
from jax import lax
from jax.experimental import pallas as pl
from jax.experimental.pallas import tpu as pltpu
from jax.experimental.pallas import tpu_sc as plsc

DIM = 96
HEADS = 4
H = W = 384
L = H * W
HW4 = L // HEADS
CPH = DIM // HEADS


_HB = 8
_NHB = H // _HB


def _qkv_body(xa_ref, xb_ref, w_ref, dw_ref, out_ref):
    x16 = jnp.concatenate([xa_ref[...], xb_ref[...]], axis=1)
    xh = x16[:, 0:_HB + 2, :]
    mm = jnp.dot(w_ref[...], xh.reshape(DIM, -1),
                 preferred_element_type=jnp.float32,
                 precision=lax.Precision.HIGHEST)
    qh = mm.reshape(DIM, _HB + 2, W + 2)
    acc = jnp.zeros((DIM, _HB, W), dtype=jnp.float32)
    for di in range(3):
        for dj in range(3):
            tap = dw_ref[:, 3 * di + dj][:, None, None]
            acc = acc + qh[:, di:di + _HB, dj:dj + W] * tap
    out_ref[...] = acc


def _dw_body(xa_ref, xb_ref, dw_ref, out_ref):
    x16 = jnp.concatenate([xa_ref[...], xb_ref[...]], axis=1)
    qh = x16[:, 0:_HB + 2, :].astype(jnp.bfloat16).astype(jnp.float32)
    dw = dw_ref[...]
    acc = jnp.zeros((DIM, _HB, W), dtype=jnp.float32)
    for di in range(3):
        for dj in range(3):
            tap = dw[:, 3 * di + dj][:, None, None]
            acc = acc + qh[:, di:di + _HB, dj:dj + W] * tap
    out_ref[...] = acc


def _dw_conv(q_pad, dw2d):
    return pl.pallas_call(
        _dw_body,
        grid=(5, _NHB),
        in_specs=[
            pl.BlockSpec((DIM, _HB, W + 2), lambda cb, hb: (cb, hb, 0)),
            pl.BlockSpec((DIM, _HB, W + 2), lambda cb, hb: (cb, hb + 1, 0)),
            pl.BlockSpec((DIM, 9), lambda cb, hb: (cb, 0)),
        ],
        out_specs=pl.BlockSpec((DIM, _HB, W), lambda cb, hb: (cb, hb, 0)),
        out_shape=jax.ShapeDtypeStruct((5 * DIM, H, W), jnp.float32),
    )(q_pad, q_pad, dw2d)


def _qkv_conv(x_pad, w2d, dw2d):
    grid = (5, _NHB)
    return pl.pallas_call(
        _qkv_body,
        grid=grid,
        in_specs=[
            pl.BlockSpec((DIM, _HB, W + 2), lambda cb, hb: (0, hb, 0)),
            pl.BlockSpec((DIM, _HB, W + 2), lambda cb, hb: (0, hb + 1, 0)),
            pl.BlockSpec((DIM, DIM), lambda cb, hb: (cb, 0)),
            pl.BlockSpec((DIM, 9), lambda cb, hb: (cb, 0)),
        ],
        out_specs=pl.BlockSpec((DIM, _HB, W), lambda cb, hb: (cb, hb, 0)),
        out_shape=jax.ShapeDtypeStruct((5 * DIM, H, W), jnp.float32),
    )(x_pad, x_pad, w2d, dw2d)


_PC = 4096
_NPC = HW4 // _PC


def _gram_body(qk_ref, g_ref):
    @pl.when(pl.program_id(1) == 0)
    def _():
        g_ref[...] = jnp.zeros_like(g_ref)
    qk = qk_ref[0]
    g_ref[0] += jnp.dot(qk, qk.T, preferred_element_type=jnp.float32)


def _gram(qk):
    return pl.pallas_call(
        _gram_body,
        grid=(HEADS, _NPC),
        in_specs=[pl.BlockSpec((1, 2 * DIM, _PC), lambda h, p: (h, 0, p))],
        out_specs=pl.BlockSpec((1, 2 * DIM, 2 * DIM), lambda h, p: (h, 0, 0)),
        out_shape=jax.ShapeDtypeStruct((HEADS, 2 * DIM, 2 * DIM), jnp.float32),
    )(qk)


def _attn_body(g_ref, t_ref, a_ref):
    g = g_ref[0]
    n = 2 * DIM
    eye = (lax.broadcasted_iota(jnp.int32, (n, n), 0)
           == lax.broadcasted_iota(jnp.int32, (n, n), 1)).astype(jnp.float32)
    diag = jnp.sum(g * eye, axis=1)
    inv = 1.0 / jnp.maximum(jnp.sqrt(diag), 1e-12)
    sim = g[:DIM, DIM:] * inv[:DIM, None] * inv[None, DIM:]
    t = t_ref[0][0:1, 0:1]
    e = jnp.exp(sim * t)
    a_ref[0] = e / (jnp.sum(e, axis=1, keepdims=True) + 1.0)


def _attn_softmax(g, temp_b):
    return pl.pallas_call(
        _attn_body,
        grid=(HEADS,),
        in_specs=[
            pl.BlockSpec((1, 2 * DIM, 2 * DIM), lambda h: (h, 0, 0)),
            pl.BlockSpec((1, 8, 128), lambda h: (h, 0, 0)),
        ],
        out_specs=pl.BlockSpec((1, DIM, DIM), lambda h: (h, 0, 0)),
        out_shape=jax.ShapeDtypeStruct((HEADS, DIM, DIM), jnp.float32),
    )(g, temp_b)


def _mix_body(a_ref, v_ref, o_ref):
    o_ref[0] = jnp.dot(a_ref[0], v_ref[0], preferred_element_type=jnp.float32)


def _mix(attn, v):
    return pl.pallas_call(
        _mix_body,
        grid=(HEADS, _NPC),
        in_specs=[
            pl.BlockSpec((1, DIM, DIM), lambda h, p: (h, 0, 0)),
            pl.BlockSpec((1, DIM, _PC), lambda h, p: (h, 0, p)),
        ],
        out_specs=pl.BlockSpec((1, DIM, _PC), lambda h, p: (h, 0, p)),
        out_shape=jax.ShapeDtypeStruct((HEADS, DIM, HW4), jnp.float32),
    )(attn, v)


_LC = 8192
_NLC = L // _LC


def _proj_body(w_ref, x_ref, o_ref):
    o_ref[...] = jnp.dot(w_ref[...], x_ref[...],
                         preferred_element_type=jnp.float32)


def _proj(w2d, x2d):
    o, i = w2d.shape
    return pl.pallas_call(
        _proj_body,
        grid=(_NLC,),
        in_specs=[
            pl.BlockSpec((o, i), lambda j: (0, 0)),
            pl.BlockSpec((i, _LC), lambda j: (0, j)),
        ],
        out_specs=pl.BlockSpec((o, _LC), lambda j: (0, j)),
        out_shape=jax.ShapeDtypeStruct((o, L), jnp.float32),
    )(w2d, x2d)


_NROW = DIM
_NSC = 2
_NTILE = 16
_CHUNK = L // _NTILE
_SUB = _CHUNK // 16
_RADIX = 256


def _sc_sort_body(v_hbm, vs_hbm, idx_hbm, rank_hbm,
                  fbuf, keybuf, ibuf, posbuf, h2, run2, b2, hgall, histb,
                  baseb, ak, ai, bk, bi, rs, hg, sem1, sem2, sem3):
    core = lax.axis_index("c")
    t = lax.axis_index("s")
    lanes = lax.iota(jnp.int32, 16)
    gidx0 = lanes * _SUB
    cbase = t * _CHUNK
    minint = jnp.int32(-2147483648)

    def zero_counts(ref):
        def zb(j, _):
            ref[pl.ds(j * 16, 16)] = jnp.zeros((16,), jnp.int32)
            return 0
        lax.fori_loop(0, _RADIX, zb, 0, unroll=8)

    def sweep_hist(sh):
        zero_counts(h2)

        def body(j, _):
            kv = plsc.load_gather(keybuf, [gidx0 + j])
            d = lax.shift_right_logical(kv, sh) & 255
            cidx = d * 16 + lanes
            h = plsc.load_gather(h2, [cidx])
            plsc.store_scatter(h2, [cidx], h + 1)
            return 0
        lax.fori_loop(0, _SUB, body, 0, unroll=8)
        def tot(g, _):
            dd = lanes + jnp.full((16,), g * 16, jnp.int32)
            def lsum(l, acc):
                return acc + plsc.load_gather(h2, [dd * 16 + l])
            acc = lax.fori_loop(0, 16, lsum, jnp.zeros((16,), jnp.int32))
            histb[pl.ds(g * 16, 16)] = acc
            return 0
        lax.fori_loop(0, _RADIX // 16, tot, 0)
        def expref(d, _):
            row = h2[pl.ds(d * 16, 16)]
            h2[pl.ds(d * 16, 16)] = plsc.cumsum(row) - row
            return 0
        lax.fori_loop(0, _RADIX, expref, 0)

    def combine():
        pltpu.sync_copy(hg, hgall)
        def dig(g, carry):
            acc = jnp.zeros((16,), jnp.int32)
            mine = jnp.zeros((16,), jnp.int32)
            def tt_body(tt, c):
                acc, mine = c
                row = hgall[tt, pl.ds(g * 16, 16)]
                mine = jnp.where(jnp.full((16,), tt, jnp.int32)
                                 < jnp.full((16,), t, jnp.int32),
                                 mine + row, mine)
                return (acc + row, mine)
            acc, mine = lax.fori_loop(0, _NTILE, tt_body, (acc, mine))
            ex = plsc.cumsum(acc) - acc + jnp.full((16,), carry, jnp.int32)
            baseb[pl.ds(g * 16, 16)] = ex + mine
            return carry + jnp.sum(acc, axis=0)
        lax.fori_loop(0, _RADIX // 16, dig, jnp.int32(0))
        def bd(g, _):
            dd = lanes + jnp.full((16,), g * 16, jnp.int32)
            bv = baseb[pl.ds(g * 16, 16)]
            def lb(l, _):
                cidx = dd * 16 + l
                plsc.store_scatter(b2, [cidx],
                                   plsc.load_gather(h2, [cidx]) + bv)
                return 0
            lax.fori_loop(0, 16, lb, 0)
            return 0
        lax.fori_loop(0, _RADIX // 16, bd, 0)

    def sweep_rank(sh):
        zero_counts(run2)

        def body(j, _):
            kv = plsc.load_gather(keybuf, [gidx0 + j])
            d = lax.shift_right_logical(kv, sh) & 255
            cidx = d * 16 + lanes
            c = plsc.load_gather(run2, [cidx])
            b = plsc.load_gather(b2, [cidx])
            plsc.store_scatter(run2, [cidx], c + 1)
            plsc.store_scatter(posbuf, [gidx0 + j], b + c)
            return 0
        lax.fori_loop(0, _SUB, body, 0, unroll=8)

    def one_pass(sh, src_k, src_i, dst_k, dst_i, first, row, last=False):
        if first:
            pltpu.sync_copy(v_hbm.at[row, pl.ds(cbase, _CHUNK)], fbuf)
            def keyb(j, _):
                x = fbuf[pl.ds(j * 16, 16)]
                k = lax.bitcast_convert_type(x, jnp.int32)
                m = lax.shift_right_arithmetic(k, 31)
                keybuf[pl.ds(j * 16, 16)] = k ^ (m | minint)
                ibuf[pl.ds(j * 16, 16)] = lanes + jnp.full(
                    (16,), cbase + j * 16, jnp.int32)
                return 0
            lax.fori_loop(0, _SUB, keyb, 0, unroll=8)
        else:
            pltpu.sync_copy(src_k.at[pl.ds(cbase, _CHUNK)], keybuf)
            pltpu.sync_copy(src_i.at[pl.ds(cbase, _CHUNK)], ibuf)
        sweep_hist(sh)
        pltpu.sync_copy(histb, hg.at[t])
        plsc.subcore_barrier()
        combine()
        sweep_rank(sh)
        d1 = pltpu.async_copy(keybuf, dst_k.at[posbuf], sem1)
        d2 = pltpu.async_copy(ibuf, dst_i.at[posbuf], sem2)
        if last:
            d3 = pltpu.async_copy(posbuf, rs.at[ibuf], sem3)
            d3.wait()
        d1.wait()
        d2.wait()
        plsc.subcore_barrier()

    def do_row(r, _):
        row = core * (_NROW // _NSC) + r
        one_pass(jnp.int32(0), None, None, bk, bi, True, row)
        one_pass(jnp.int32(8), bk, bi, ak, ai, False, row)
        one_pass(jnp.int32(16), ak, ai, bk, bi, False, row)
        one_pass(jnp.int32(24), bk, bi, ak, ai, False, row, last=True)
        pltpu.sync_copy(ak.at[pl.ds(cbase, _CHUNK)], keybuf)
        pltpu.sync_copy(ai.at[pl.ds(cbase, _CHUNK)], ibuf)
        def unk(j, _):
            kv = keybuf[pl.ds(j * 16, 16)]
            m = lax.shift_right_arithmetic(kv, 31)
            orig = kv ^ jnp.where(m != 0, minint, jnp.int32(-1))
            fbuf[pl.ds(j * 16, 16)] = lax.bitcast_convert_type(orig, jnp.float32)
            return 0
        lax.fori_loop(0, _SUB, unk, 0, unroll=8)
        pltpu.sync_copy(fbuf, vs_hbm.at[row, pl.ds(cbase, _CHUNK)])
        pltpu.sync_copy(ibuf, idx_hbm.at[row, pl.ds(cbase, _CHUNK)])
        pltpu.sync_copy(rs.at[pl.ds(cbase, _CHUNK)], keybuf)
        pltpu.sync_copy(keybuf, rank_hbm.at[row, pl.ds(cbase, _CHUNK)])
        plsc.subcore_barrier()
        return 0

    lax.fori_loop(0, _NROW // _NSC, do_row, 0)


def _sc_argsort(v):
    mesh = plsc.VectorSubcoreMesh(core_axis_name="c", subcore_axis_name="s")
    f = pl.kernel(
        _sc_sort_body,
        out_type=(jax.ShapeDtypeStruct((_NROW, L), jnp.float32),
                  jax.ShapeDtypeStruct((_NROW, L), jnp.int32),
                  jax.ShapeDtypeStruct((_NROW, L), jnp.int32)),
        mesh=mesh,
        compiler_params=pltpu.CompilerParams(needs_layout_passes=False),
        scratch_types=[
            pltpu.VMEM((_CHUNK,), jnp.float32),
            pltpu.VMEM((_CHUNK,), jnp.int32),
            pltpu.VMEM((_CHUNK,), jnp.int32),
            pltpu.VMEM((_CHUNK,), jnp.int32),
            pltpu.VMEM((_RADIX * 16,), jnp.int32),
            pltpu.VMEM((_RADIX * 16,), jnp.int32),
            pltpu.VMEM((_RADIX * 16,), jnp.int32),
            pltpu.VMEM((_NTILE, _RADIX), jnp.int32),
            pltpu.VMEM((_RADIX,), jnp.int32),
            pltpu.VMEM((_RADIX,), jnp.int32),
            pltpu.VMEM_SHARED((L,), jnp.int32),
            pltpu.VMEM_SHARED((L,), jnp.int32),
            pltpu.VMEM_SHARED((L,), jnp.int32),
            pltpu.VMEM_SHARED((L,), jnp.int32),
            pltpu.VMEM_SHARED((L,), jnp.int32),
            pltpu.VMEM_SHARED((_NTILE, _RADIX), jnp.int32),
            pltpu.SemaphoreType.DMA,
            pltpu.SemaphoreType.DMA,
            pltpu.SemaphoreType.DMA,
        ])
    return f(v)


_HC = DIM // 2


def _sc_inv_body(iw_hbm, ih_hbm, comp_hbm, ibuf, dbuf, sbuf, obuf,
                 slw, slh, sem):
    core = lax.axis_index("c")
    t = lax.axis_index("s")
    lanes = lax.iota(jnp.int32, 16)

    def do_c(cc, _):
        c = core * (_HC // _NSC) + cc
        pltpu.sync_copy(iw_hbm.at[c, pl.ds(t * 24, 24), :], ibuf)

        def rw(r, _):
            hrow = t * 24 + r
            def kk(k, _):
                iv = ibuf[r, pl.ds(k * 16, 16)]
                off = (r * 24 + k) * 16
                dbuf[pl.ds(off, 16)] = iv + jnp.full((16,), hrow * W,
                                                     jnp.int32)
                sbuf[pl.ds(off, 16)] = lanes + jnp.full((16,), k * 16,
                                                        jnp.int32)
                return 0
            lax.fori_loop(0, 24, kk, 0)
            return 0
        lax.fori_loop(0, 24, rw, 0)
        pltpu.async_copy(sbuf, slw.at[dbuf], sem).wait()
        pltpu.sync_copy(ih_hbm.at[c, pl.ds(t * 24, 24), :], ibuf)

        def rh(r, _):
            hrow = t * 24 + r
            def kk(k, _):
                iv = ibuf[r, pl.ds(k * 16, 16)]
                off = (r * 24 + k) * 16
                dbuf[pl.ds(off, 16)] = iv * W + lanes + jnp.full(
                    (16,), k * 16, jnp.int32)
                sbuf[pl.ds(off, 16)] = jnp.full((16,), hrow, jnp.int32)
                return 0
            lax.fori_loop(0, 24, kk, 0)
            return 0
        lax.fori_loop(0, 24, rh, 0)
        pltpu.async_copy(sbuf, slh.at[dbuf], sem).wait()
        plsc.subcore_barrier()
        pltpu.sync_copy(slh.at[pl.ds(t * _CHUNK, _CHUNK)], obuf)

        def rc(r, _):
            def kk(k, _):
                off = (r * 24 + k) * 16
                rhv = obuf[pl.ds(off, 16)]
                dbuf[pl.ds(off, 16)] = rhv * W + lanes + jnp.full(
                    (16,), k * 16, jnp.int32)
                return 0
            lax.fori_loop(0, 24, kk, 0)
            return 0
        lax.fori_loop(0, 24, rc, 0)
        pltpu.async_copy(slw.at[dbuf], sbuf, sem).wait()

        def rc2(r, _):
            def kk(k, _):
                off = (r * 24 + k) * 16
                sbuf[pl.ds(off, 16)] = obuf[pl.ds(off, 16)] * W + sbuf[
                    pl.ds(off, 16)]
                return 0
            lax.fori_loop(0, 24, kk, 0)
            return 0
        lax.fori_loop(0, 24, rc2, 0)
        pltpu.sync_copy(sbuf, comp_hbm.at[c, pl.ds(t * _CHUNK, _CHUNK)])
        plsc.subcore_barrier()
        return 0

    lax.fori_loop(0, _HC // _NSC, do_c, 0)


def _sc_invert(idx_w, idx_h):
    mesh = plsc.VectorSubcoreMesh(core_axis_name="c", subcore_axis_name="s")
    f = pl.kernel(
        _sc_inv_body,
        out_type=jax.ShapeDtypeStruct((_HC, L), jnp.int32),
        mesh=mesh,
        compiler_params=pltpu.CompilerParams(needs_layout_passes=False),
        scratch_types=[
            pltpu.VMEM((24, W), jnp.int32),
            pltpu.VMEM((_CHUNK,), jnp.int32),
            pltpu.VMEM((_CHUNK,), jnp.int32),
            pltpu.VMEM((_CHUNK,), jnp.int32),
            pltpu.VMEM_SHARED((L,), jnp.int32),
            pltpu.VMEM_SHARED((L,), jnp.int32),
            pltpu.SemaphoreType.DMA,
        ])
    return f(idx_w, idx_h)


def _scatter_axis(idx, vals, axis):
    grids = list(jnp.indices(idx.shape))
    grids[axis] = idx
    return jnp.zeros_like(vals).at[tuple(grids)].set(vals)


def _fold_box(t):
    return t.reshape(HEADS, CPH, HEADS, HW4).reshape(HEADS, DIM, HW4)


def _unfold_box(t):
    return t.reshape(HEADS, CPH, HEADS, HW4).reshape(DIM, L)


def _perm_nb():
    return (jnp.arange(HW4, dtype=jnp.int32)[None, :] * HEADS
            + jnp.arange(HEADS, dtype=jnp.int32)[:, None]).reshape(L)


def _iperm_nb():
    return (jnp.arange(HEADS, dtype=jnp.int32)[None, :] * HW4
            + jnp.arange(HW4, dtype=jnp.int32)[:, None]).reshape(L)


def kernel(x, w_qkv, w_dw, w_out, temperature):
    xs = x[0]
    half = DIM // 2

    xh = xs[:half]
    idx_h = jnp.argsort(xh, axis=-2)
    x_sort = jnp.take_along_axis(xh, idx_h, axis=-2)
    idx_w = jnp.argsort(x_sort, axis=-1)
    x_sort = jnp.take_along_axis(x_sort, idx_w, axis=-1)
    xs = xs.at[:half].set(x_sort)

    _c = _proj(w_qkv[:, :, 0, 0], xs.reshape(DIM, L)).reshape(5 * DIM, H, W)
    _cp = jnp.pad(_c, ((0, 0), (1, 7), (1, 1)))
    qkv = _dw_conv(_cp, w_dw.reshape(5 * DIM, 9))
    q1, k1, q2, k2, v = jnp.split(qkv.reshape(5, DIM, L), 5, axis=0)
    q1, k1, q2, k2, v = q1[0], k1[0], q2[0], k2[0], v[0]

    vs, idx, rank = _sc_argsort(v)
    idx2 = jnp.take(idx, _perm_nb(), axis=1)
    g = lambda t: jnp.take_along_axis(t, idx, axis=-1)
    g2 = lambda t: jnp.take_along_axis(t, idx2, axis=-1)
    q1s, k1s = g(q1), g(k1)
    q2s_nb, k2s_nb, vs_nb = g2(q2), g2(k2), g2(v)

    temp_b = jnp.broadcast_to(temperature.reshape(HEADS, 1, 1), (HEADS, 8, 128))

    qk1 = jnp.concatenate([_fold_box(q1s), _fold_box(k1s)], axis=1)
    attn1 = _attn_softmax(_gram(qk1), temp_b)
    out1 = _mix(attn1, _fold_box(vs))

    qk2 = jnp.concatenate([_fold_box(q2s_nb), _fold_box(k2s_nb)], axis=1)
    attn2 = _attn_softmax(_gram(qk2), temp_b)
    out2 = _mix(attn2, _fold_box(vs_nb))

    out2n = jnp.take(_unfold_box(out2), _iperm_nb(), axis=1)
    prod = _unfold_box(out1) * out2n
    res = jnp.take_along_axis(prod, rank, axis=-1)

    out = _proj(w_out[:, :, 0, 0], res)

    comp = _sc_invert(idx_w.astype(jnp.int32), idx_h.astype(jnp.int32))
    first = jnp.take_along_axis(out[:half], comp, axis=-1)
    out = jnp.concatenate([first, out[half:]], axis=0)
    return out.reshape(DIM, H, W)[None]
